# baseline (speedup 1.0000x reference)
import functools

import jax
import jax.numpy as jnp
from jax import lax
from jax.experimental import pallas as pl
from jax.experimental.pallas import tpu as pltpu
from jax.experimental.pallas import tpu_sc as plsc

N_NODES = 10000
N_EDGES = 320000
D_IN = 128
HID = 128
D_OUT = 64
N_GROUPS = 16

NC = 2
NS = 16
NW = NC * NS
LANES = 16

N_PAD = 10240
ROWS_PER_TILE = N_PAD // NS
E_PAD = 327680
EPW = E_PAD // NW
KW = 128
NWIN = EPW // KW
CH = 16
NCH = NWIN // CH
N_DUMP = N_PAD - N_NODES

ROW_BLK = 1024
N_BLKS = N_PAD // ROW_BLK

_f32 = jnp.float32
_i32 = jnp.int32


def _sc_mesh():
    return plsc.VectorSubcoreMesh(
        core_axis_name="c", subcore_axis_name="s", num_cores=NC, num_subcores=NS
    )


def _zero_vec_ref(ref, nvecs):
    zeros16 = jnp.zeros((LANES,), _f32)

    def body(i, _):
        ref[pl.ds(i * LANES, LANES)] = zeros16
        return 0

    lax.fori_loop(0, nvecs, body, 0)


def _deg_body(didx_hbm, out_hbm, didx_v, ones_v, zbuf_v, accd_sh, sem):
    del sem
    c = lax.axis_index("c")
    s = lax.axis_index("s")
    wid = c * NS + s

    pltpu.sync_copy(didx_hbm.at[wid], didx_v)

    ones16 = jnp.ones((LANES,), _f32)

    def fill_ones(i, _):
        ones_v[pl.ds(i * LANES, LANES)] = ones16
        return 0

    lax.fori_loop(0, KW // LANES, fill_ones, 0)
    _zero_vec_ref(zbuf_v, ROWS_PER_TILE // LANES)
    pltpu.sync_copy(zbuf_v, accd_sh.at[pl.ds(s * ROWS_PER_TILE, ROWS_PER_TILE)])
    plsc.subcore_barrier()

    def win(j, _):
        pltpu.sync_copy(ones_v, accd_sh.at[didx_v.at[j]], add=True)
        return 0

    lax.fori_loop(0, NWIN, win, 0)
    plsc.subcore_barrier()

    pltpu.sync_copy(
        accd_sh.at[pl.ds(s * ROWS_PER_TILE, ROWS_PER_TILE)],
        out_hbm.at[c].at[pl.ds(s * ROWS_PER_TILE, ROWS_PER_TILE)],
    )


def _sc_degree(didx_r):
    k = pl.kernel(
        _deg_body,
        out_type=jax.ShapeDtypeStruct((NC, N_PAD), _f32),
        mesh=_sc_mesh(),
        scratch_types=[
            pltpu.VMEM((NWIN, KW), _i32),
            pltpu.VMEM((KW,), _f32),
            pltpu.VMEM((ROWS_PER_TILE,), _f32),
            pltpu.VMEM_SHARED((N_PAD,), _f32),
            pltpu.SemaphoreType.DMA,
        ],
        name="gcn_degree_sc",
    )
    return k(didx_r)


def _scat_body(hws_hbm, sidx_hbm, didx_hbm, out_hbm,
               sidx_v, didx_v, rows_v, acc_sh, sem0, sem1):
    c = lax.axis_index("c")
    s = lax.axis_index("s")
    wid = c * NS + s

    zrow = rows_v.at[0]

    def zrow_fill(i, _):
        zrow[i // (HID // LANES),
             pl.ds((i % (HID // LANES)) * LANES, LANES)] = jnp.zeros((LANES,), _f32)
        return 0

    lax.fori_loop(0, KW * HID // LANES, zrow_fill, 0)

    def zcopy(i, _):
        pltpu.sync_copy(zrow, acc_sh.at[pl.ds(s * ROWS_PER_TILE + i * KW, KW)])
        return 0

    lax.fori_loop(0, ROWS_PER_TILE // KW, zcopy, 0)
    plsc.subcore_barrier()

    sems = (sem0, sem1)

    def chunk(ch, _):
        pltpu.sync_copy(sidx_hbm.at[wid].at[pl.ds(ch * CH, CH)], sidx_v)
        pltpu.sync_copy(didx_hbm.at[wid].at[pl.ds(ch * CH, CH)], didx_v)

        pltpu.async_copy(hws_hbm.at[sidx_v.at[0]], rows_v.at[0], sem0)
        pltpu.async_copy(hws_hbm.at[sidx_v.at[1]], rows_v.at[1], sem1)

        def win(w, _):
            for b in range(2):
                j = w * 2 + b
                buf = rows_v.at[b]
                pltpu.make_async_copy(hws_hbm.at[sidx_v.at[j]], buf, sems[b]).wait()
                pltpu.sync_copy(buf, acc_sh.at[didx_v.at[j]], add=True)

                @pl.when(j + 2 < CH)
                def _():
                    pltpu.async_copy(hws_hbm.at[sidx_v.at[j + 2]], buf, sems[b])
            return 0

        lax.fori_loop(0, CH // 2, win, 0)
        return 0

    lax.fori_loop(0, NCH, chunk, 0)
    plsc.subcore_barrier()

    pltpu.sync_copy(
        acc_sh.at[pl.ds(s * ROWS_PER_TILE, ROWS_PER_TILE)],
        out_hbm.at[c].at[pl.ds(s * ROWS_PER_TILE, ROWS_PER_TILE)],
    )


def _sc_gather_scatter(hws, sidx_r, didx_r):
    k = pl.kernel(
        _scat_body,
        out_type=jax.ShapeDtypeStruct((NC, N_PAD, HID), _f32),
        mesh=_sc_mesh(),
        scratch_types=[
            pltpu.VMEM((CH, KW), _i32),
            pltpu.VMEM((CH, KW), _i32),
            pltpu.VMEM((2, KW, HID), _f32),
            pltpu.VMEM_SHARED((N_PAD, HID), _f32),
            pltpu.SemaphoreType.DMA,
            pltpu.SemaphoreType.DMA,
        ],
        name="gcn_gather_scatter_sc",
    )
    return k(hws, sidx_r, didx_r)


def _dinv_from_pdeg(pdeg_ref):
    deg = pdeg_ref[:, 0:1] + pdeg_ref[:, 1:2] + 1.0
    return 1.0 / jnp.sqrt(deg)


def _encode_kernel(x_ref, win_ref, bin_ref, w0_ref, pdeg_ref, out_ref):
    h0 = jnp.dot(x_ref[...], win_ref[...], preferred_element_type=_f32) + bin_ref[...]
    hw0 = jnp.dot(h0, w0_ref[...], preferred_element_type=_f32)
    out_ref[...] = _dinv_from_pdeg(pdeg_ref) * hw0


def _tc_encode(x_pad, w_in, b_in, w0, pdeg_t):
    return pl.pallas_call(
        _encode_kernel,
        grid=(N_BLKS,),
        out_shape=jax.ShapeDtypeStruct((N_PAD, HID), _f32),
        in_specs=[
            pl.BlockSpec((ROW_BLK, D_IN), lambda i: (i, 0)),
            pl.BlockSpec((D_IN, HID), lambda i: (0, 0)),
            pl.BlockSpec((HID,), lambda i: (0,)),
            pl.BlockSpec((HID, HID), lambda i: (0, 0)),
            pl.BlockSpec((ROW_BLK, NC), lambda i: (i, 0)),
        ],
        out_specs=pl.BlockSpec((ROW_BLK, HID), lambda i: (i, 0)),
    )(x_pad, w_in, b_in, w0, pdeg_t)


def _mid_kernel(acc_ref, hws_ref, pdeg_ref, b_ref, w_ref, out_ref):
    dinv = _dinv_from_pdeg(pdeg_ref)
    t = acc_ref[0] + acc_ref[1] + hws_ref[...]
    h = jnp.maximum(dinv * t + b_ref[...], 0.0)
    out_ref[...] = dinv * jnp.dot(h, w_ref[...], preferred_element_type=_f32)


def _tc_mid(acc, hws, pdeg_t, b, w):
    return pl.pallas_call(
        _mid_kernel,
        grid=(N_BLKS,),
        out_shape=jax.ShapeDtypeStruct((N_PAD, HID), _f32),
        in_specs=[
            pl.BlockSpec((NC, ROW_BLK, HID), lambda i: (0, i, 0)),
            pl.BlockSpec((ROW_BLK, HID), lambda i: (i, 0)),
            pl.BlockSpec((ROW_BLK, NC), lambda i: (i, 0)),
            pl.BlockSpec((HID,), lambda i: (0,)),
            pl.BlockSpec((HID, HID), lambda i: (0, 0)),
        ],
        out_specs=pl.BlockSpec((ROW_BLK, HID), lambda i: (i, 0)),
    )(acc, hws, pdeg_t, b, w)


def _final_kernel(acc_ref, hws_ref, pdeg_ref, b_ref, wcls_ref, bcls_ref, bid_ref,
                  out_ref, sums_acc, cnts_acc):
    i = pl.program_id(0)

    @pl.when(i == 0)
    def _():
        sums_acc[...] = jnp.zeros_like(sums_acc)
        cnts_acc[...] = jnp.zeros_like(cnts_acc)

    t = acc_ref[0] + acc_ref[1] + hws_ref[...]
    h = _dinv_from_pdeg(pdeg_ref) * t + b_ref[...]
    y = jnp.dot(h, wcls_ref[...], preferred_element_type=_f32) + bcls_ref[...]

    bid = bid_ref[0]
    iota = lax.broadcasted_iota(_i32, (N_GROUPS, ROW_BLK), 0)
    mask = (bid == iota).astype(_f32)
    sums_acc[...] += jnp.dot(mask, y, preferred_element_type=_f32)
    cnts_acc[...] += jnp.broadcast_to(
        jnp.sum(mask, axis=1, keepdims=True), (N_GROUPS, D_OUT)
    )

    @pl.when(i == N_BLKS - 1)
    def _():
        out_ref[...] = sums_acc[...] / jnp.maximum(cnts_acc[...], 1.0)


def _tc_final(acc, hws, pdeg_t, b2, w_cls, b_cls, bid3):
    return pl.pallas_call(
        _final_kernel,
        grid=(N_BLKS,),
        out_shape=jax.ShapeDtypeStruct((N_GROUPS, D_OUT), _f32),
        in_specs=[
            pl.BlockSpec((NC, ROW_BLK, HID), lambda i: (0, i, 0)),
            pl.BlockSpec((ROW_BLK, HID), lambda i: (i, 0)),
            pl.BlockSpec((ROW_BLK, NC), lambda i: (i, 0)),
            pl.BlockSpec((HID,), lambda i: (0,)),
            pl.BlockSpec((HID, D_OUT), lambda i: (0, 0)),
            pl.BlockSpec((D_OUT,), lambda i: (0,)),
            pl.BlockSpec((1, 1, ROW_BLK), lambda i: (i, 0, 0)),
        ],
        out_specs=pl.BlockSpec((N_GROUPS, D_OUT), lambda i: (0, 0)),
        scratch_shapes=[
            pltpu.VMEM((N_GROUPS, D_OUT), _f32),
            pltpu.VMEM((N_GROUPS, D_OUT), _f32),
        ],
    )(acc, hws, pdeg_t, b2, w_cls, b_cls, bid3)


def kernel(x, edge_index, batch_ids, W_in, b_in, W0, b0, W1, b1, W2, b2,
           W_cls, b_cls):
    n_extra = E_PAD - N_EDGES
    sidx_pad = jnp.concatenate(
        [edge_index[0], (jnp.arange(n_extra, dtype=_i32) % N_NODES)])
    didx_pad = jnp.concatenate(
        [edge_index[1], N_NODES + (jnp.arange(n_extra, dtype=_i32) % N_DUMP)])
    sidx_r = sidx_pad.reshape(NW, NWIN, KW)
    didx_r = didx_pad.reshape(NW, NWIN, KW)

    x_pad = jnp.pad(x, ((0, N_PAD - N_NODES), (0, 0)))
    bid3 = jnp.pad(batch_ids, (0, N_PAD - N_NODES),
                   constant_values=N_GROUPS).reshape(N_BLKS, 1, ROW_BLK)

    pdeg_t = _sc_degree(didx_r).T

    hws0 = _tc_encode(x_pad, W_in, b_in, W0, pdeg_t)
    acc0 = _sc_gather_scatter(hws0, sidx_r, didx_r)
    hws1 = _tc_mid(acc0, hws0, pdeg_t, b0, W1)
    acc1 = _sc_gather_scatter(hws1, sidx_r, didx_r)
    hws2 = _tc_mid(acc1, hws1, pdeg_t, b1, W2)
    acc2 = _sc_gather_scatter(hws2, sidx_r, didx_r)
    return _tc_final(acc2, hws2, pdeg_t, b2, W_cls, b_cls, bid3)


# speedup vs baseline: 1.0841x; 1.0841x over previous
import functools

import jax
import jax.numpy as jnp
from jax import lax
from jax.experimental import pallas as pl
from jax.experimental.pallas import tpu as pltpu
from jax.experimental.pallas import tpu_sc as plsc

N_NODES = 10000
N_EDGES = 320000
D_IN = 128
HID = 128
D_OUT = 64
N_GROUPS = 16

NC = 2
NS = 16
NW = NC * NS
LANES = 16

N_PAD = 10240
ROWS_PER_TILE = N_PAD // NS
E_PAD = 327680
EPW = E_PAD // NW
KW = 64
NWIN = EPW // KW
CH = 32
NCH = NWIN // CH
SLOTS = 4
N_DUMP = N_PAD - N_NODES

ROW_BLK = 1024
N_BLKS = N_PAD // ROW_BLK

_f32 = jnp.float32
_i32 = jnp.int32


def _sc_mesh():
    return plsc.VectorSubcoreMesh(
        core_axis_name="c", subcore_axis_name="s", num_cores=NC, num_subcores=NS
    )


def _zero_vec_ref(ref, nvecs):
    zeros16 = jnp.zeros((LANES,), _f32)

    def body(i, _):
        ref[pl.ds(i * LANES, LANES)] = zeros16
        return 0

    lax.fori_loop(0, nvecs, body, 0)


def _deg_body(didx_hbm, out_hbm, didx_v, ones_v, zbuf_v, accd_sh, sem):
    del sem
    c = lax.axis_index("c")
    s = lax.axis_index("s")
    wid = c * NS + s

    pltpu.sync_copy(didx_hbm.at[wid], didx_v)

    ones16 = jnp.ones((LANES,), _f32)

    def fill_ones(i, _):
        ones_v[pl.ds(i * LANES, LANES)] = ones16
        return 0

    lax.fori_loop(0, KW // LANES, fill_ones, 0)
    _zero_vec_ref(zbuf_v, ROWS_PER_TILE // LANES)
    pltpu.sync_copy(zbuf_v, accd_sh.at[pl.ds(s * ROWS_PER_TILE, ROWS_PER_TILE)])
    plsc.subcore_barrier()

    def win(j, _):
        pltpu.sync_copy(ones_v, accd_sh.at[didx_v.at[j]], add=True)
        return 0

    lax.fori_loop(0, NWIN, win, 0)
    plsc.subcore_barrier()

    pltpu.sync_copy(
        accd_sh.at[pl.ds(s * ROWS_PER_TILE, ROWS_PER_TILE)],
        out_hbm.at[c].at[pl.ds(s * ROWS_PER_TILE, ROWS_PER_TILE)],
    )


def _sc_degree(didx_r):
    k = pl.kernel(
        _deg_body,
        out_type=jax.ShapeDtypeStruct((NC, N_PAD), _f32),
        mesh=_sc_mesh(),
        scratch_types=[
            pltpu.VMEM((NWIN, KW), _i32),
            pltpu.VMEM((KW,), _f32),
            pltpu.VMEM((ROWS_PER_TILE,), _f32),
            pltpu.VMEM_SHARED((N_PAD,), _f32),
            pltpu.SemaphoreType.DMA,
        ],
        name="gcn_degree_sc",
    )
    return k(didx_r)


def _scat_body(hws_hbm, sidx_hbm, didx_hbm, out_hbm,
               sidx_v, didx_v, rows_v, acc_sh, sem0, sem1, sem2, sem3):
    c = lax.axis_index("c")
    s = lax.axis_index("s")
    wid = c * NS + s

    zrow = rows_v.at[0]

    def zrow_fill(i, _):
        zrow[i // (HID // LANES),
             pl.ds((i % (HID // LANES)) * LANES, LANES)] = jnp.zeros((LANES,), _f32)
        return 0

    lax.fori_loop(0, KW * HID // LANES, zrow_fill, 0)

    def zcopy(i, _):
        pltpu.sync_copy(zrow, acc_sh.at[pl.ds(s * ROWS_PER_TILE + i * KW, KW)])
        return 0

    lax.fori_loop(0, ROWS_PER_TILE // KW, zcopy, 0)
    plsc.subcore_barrier()

    sems = (sem0, sem1, sem2, sem3)

    def chunk(ch, _):
        pltpu.sync_copy(sidx_hbm.at[wid].at[pl.ds(ch * CH, CH)], sidx_v)
        pltpu.sync_copy(didx_hbm.at[wid].at[pl.ds(ch * CH, CH)], didx_v)

        for b in range(SLOTS):
            pltpu.async_copy(hws_hbm.at[sidx_v.at[b]], rows_v.at[b], sems[b])

        def win(w, _):
            for b in range(SLOTS):
                j = w * SLOTS + b
                buf = rows_v.at[b]
                pltpu.make_async_copy(hws_hbm.at[sidx_v.at[j]], buf, sems[b]).wait()
                pltpu.sync_copy(buf, acc_sh.at[didx_v.at[j]], add=True)

                @pl.when(j + SLOTS < CH)
                def _():
                    pltpu.async_copy(hws_hbm.at[sidx_v.at[j + SLOTS]], buf, sems[b])
            return 0

        lax.fori_loop(0, CH // SLOTS, win, 0)
        return 0

    lax.fori_loop(0, NCH, chunk, 0)
    plsc.subcore_barrier()

    pltpu.sync_copy(
        acc_sh.at[pl.ds(s * ROWS_PER_TILE, ROWS_PER_TILE)],
        out_hbm.at[c].at[pl.ds(s * ROWS_PER_TILE, ROWS_PER_TILE)],
    )


def _sc_gather_scatter(hws, sidx_r, didx_r):
    k = pl.kernel(
        _scat_body,
        out_type=jax.ShapeDtypeStruct((NC, N_PAD, HID), _f32),
        mesh=_sc_mesh(),
        scratch_types=[
            pltpu.VMEM((CH, KW), _i32),
            pltpu.VMEM((CH, KW), _i32),
            pltpu.VMEM((SLOTS, KW, HID), _f32),
            pltpu.VMEM_SHARED((N_PAD, HID), _f32),
            pltpu.SemaphoreType.DMA,
            pltpu.SemaphoreType.DMA,
            pltpu.SemaphoreType.DMA,
            pltpu.SemaphoreType.DMA,
        ],
        name="gcn_gather_scatter_sc",
    )
    return k(hws, sidx_r, didx_r)


def _dinv_from_pdeg(pdeg_ref):
    deg = pdeg_ref[:, 0:1] + pdeg_ref[:, 1:2] + 1.0
    return 1.0 / jnp.sqrt(deg)


def _encode_kernel(x_ref, win_ref, bin_ref, w0_ref, pdeg_ref, out_ref):
    h0 = jnp.dot(x_ref[...], win_ref[...], preferred_element_type=_f32) + bin_ref[...]
    hw0 = jnp.dot(h0, w0_ref[...], preferred_element_type=_f32)
    out_ref[...] = _dinv_from_pdeg(pdeg_ref) * hw0


def _tc_encode(x_pad, w_in, b_in, w0, pdeg_t):
    return pl.pallas_call(
        _encode_kernel,
        grid=(N_BLKS,),
        out_shape=jax.ShapeDtypeStruct((N_PAD, HID), _f32),
        in_specs=[
            pl.BlockSpec((ROW_BLK, D_IN), lambda i: (i, 0)),
            pl.BlockSpec((D_IN, HID), lambda i: (0, 0)),
            pl.BlockSpec((HID,), lambda i: (0,)),
            pl.BlockSpec((HID, HID), lambda i: (0, 0)),
            pl.BlockSpec((ROW_BLK, NC), lambda i: (i, 0)),
        ],
        out_specs=pl.BlockSpec((ROW_BLK, HID), lambda i: (i, 0)),
    )(x_pad, w_in, b_in, w0, pdeg_t)


def _mid_kernel(acc_ref, hws_ref, pdeg_ref, b_ref, w_ref, out_ref):
    dinv = _dinv_from_pdeg(pdeg_ref)
    t = acc_ref[0] + acc_ref[1] + hws_ref[...]
    h = jnp.maximum(dinv * t + b_ref[...], 0.0)
    out_ref[...] = dinv * jnp.dot(h, w_ref[...], preferred_element_type=_f32)


def _tc_mid(acc, hws, pdeg_t, b, w):
    return pl.pallas_call(
        _mid_kernel,
        grid=(N_BLKS,),
        out_shape=jax.ShapeDtypeStruct((N_PAD, HID), _f32),
        in_specs=[
            pl.BlockSpec((NC, ROW_BLK, HID), lambda i: (0, i, 0)),
            pl.BlockSpec((ROW_BLK, HID), lambda i: (i, 0)),
            pl.BlockSpec((ROW_BLK, NC), lambda i: (i, 0)),
            pl.BlockSpec((HID,), lambda i: (0,)),
            pl.BlockSpec((HID, HID), lambda i: (0, 0)),
        ],
        out_specs=pl.BlockSpec((ROW_BLK, HID), lambda i: (i, 0)),
    )(acc, hws, pdeg_t, b, w)


def _final_kernel(acc_ref, hws_ref, pdeg_ref, b_ref, wcls_ref, bcls_ref, bid_ref,
                  out_ref, sums_acc, cnts_acc):
    i = pl.program_id(0)

    @pl.when(i == 0)
    def _():
        sums_acc[...] = jnp.zeros_like(sums_acc)
        cnts_acc[...] = jnp.zeros_like(cnts_acc)

    t = acc_ref[0] + acc_ref[1] + hws_ref[...]
    h = _dinv_from_pdeg(pdeg_ref) * t + b_ref[...]
    y = jnp.dot(h, wcls_ref[...], preferred_element_type=_f32) + bcls_ref[...]

    bid = bid_ref[0]
    iota = lax.broadcasted_iota(_i32, (N_GROUPS, ROW_BLK), 0)
    mask = (bid == iota).astype(_f32)
    sums_acc[...] += jnp.dot(mask, y, preferred_element_type=_f32)
    cnts_acc[...] += jnp.broadcast_to(
        jnp.sum(mask, axis=1, keepdims=True), (N_GROUPS, D_OUT)
    )

    @pl.when(i == N_BLKS - 1)
    def _():
        out_ref[...] = sums_acc[...] / jnp.maximum(cnts_acc[...], 1.0)


def _tc_final(acc, hws, pdeg_t, b2, w_cls, b_cls, bid3):
    return pl.pallas_call(
        _final_kernel,
        grid=(N_BLKS,),
        out_shape=jax.ShapeDtypeStruct((N_GROUPS, D_OUT), _f32),
        in_specs=[
            pl.BlockSpec((NC, ROW_BLK, HID), lambda i: (0, i, 0)),
            pl.BlockSpec((ROW_BLK, HID), lambda i: (i, 0)),
            pl.BlockSpec((ROW_BLK, NC), lambda i: (i, 0)),
            pl.BlockSpec((HID,), lambda i: (0,)),
            pl.BlockSpec((HID, D_OUT), lambda i: (0, 0)),
            pl.BlockSpec((D_OUT,), lambda i: (0,)),
            pl.BlockSpec((1, 1, ROW_BLK), lambda i: (i, 0, 0)),
        ],
        out_specs=pl.BlockSpec((N_GROUPS, D_OUT), lambda i: (0, 0)),
        scratch_shapes=[
            pltpu.VMEM((N_GROUPS, D_OUT), _f32),
            pltpu.VMEM((N_GROUPS, D_OUT), _f32),
        ],
    )(acc, hws, pdeg_t, b2, w_cls, b_cls, bid3)


def kernel(x, edge_index, batch_ids, W_in, b_in, W0, b0, W1, b1, W2, b2,
           W_cls, b_cls):
    n_extra = E_PAD - N_EDGES
    sidx_pad = jnp.concatenate(
        [edge_index[0], (jnp.arange(n_extra, dtype=_i32) % N_NODES)])
    didx_pad = jnp.concatenate(
        [edge_index[1], N_NODES + (jnp.arange(n_extra, dtype=_i32) % N_DUMP)])
    sidx_r = sidx_pad.reshape(NW, NWIN, KW)
    didx_r = didx_pad.reshape(NW, NWIN, KW)

    x_pad = jnp.pad(x, ((0, N_PAD - N_NODES), (0, 0)))
    bid3 = jnp.pad(batch_ids, (0, N_PAD - N_NODES),
                   constant_values=N_GROUPS).reshape(N_BLKS, 1, ROW_BLK)

    pdeg_t = _sc_degree(didx_r).T

    hws0 = _tc_encode(x_pad, W_in, b_in, W0, pdeg_t)
    acc0 = _sc_gather_scatter(hws0, sidx_r, didx_r)
    hws1 = _tc_mid(acc0, hws0, pdeg_t, b0, W1)
    acc1 = _sc_gather_scatter(hws1, sidx_r, didx_r)
    hws2 = _tc_mid(acc1, hws1, pdeg_t, b1, W2)
    acc2 = _sc_gather_scatter(hws2, sidx_r, didx_r)
    return _tc_final(acc2, hws2, pdeg_t, b2, W_cls, b_cls, bid3)


# speedup vs baseline: 1.1028x; 1.0173x over previous
import functools

import jax
import jax.numpy as jnp
from jax import lax
from jax.experimental import pallas as pl
from jax.experimental.pallas import tpu as pltpu
from jax.experimental.pallas import tpu_sc as plsc

N_NODES = 10000
N_EDGES = 320000
D_IN = 128
HID = 128
D_OUT = 64
N_GROUPS = 16

NC = 2
NS = 16
NW = NC * NS
LANES = 16

N_PAD = 10240
ROWS_PER_TILE = N_PAD // NS
E_PAD = 327680
EPW = E_PAD // NW
KW = 64
NWIN = EPW // KW
CH = 32
NCH = NWIN // CH
SLOTS = 4
IW = 128
NROW = EPW // IW
N_DUMP = N_PAD - N_NODES

ROW_BLK = 1024
N_BLKS = N_PAD // ROW_BLK

_f32 = jnp.float32
_i32 = jnp.int32


def _sc_mesh():
    return plsc.VectorSubcoreMesh(
        core_axis_name="c", subcore_axis_name="s", num_cores=NC, num_subcores=NS
    )


def _zero_vec_ref(ref, nvecs):
    zeros16 = jnp.zeros((LANES,), _f32)

    def body(i, _):
        ref[pl.ds(i * LANES, LANES)] = zeros16
        return 0

    lax.fori_loop(0, nvecs, body, 0)


def _deg_body(didx_hbm, out_hbm, didx_v, ones_v, zbuf_v, accd_sh, sem):
    del sem
    c = lax.axis_index("c")
    s = lax.axis_index("s")
    wid = c * NS + s

    pltpu.sync_copy(didx_hbm.at[wid], didx_v)

    ones16 = jnp.ones((LANES,), _f32)

    def fill_ones(i, _):
        ones_v[pl.ds(i * LANES, LANES)] = ones16
        return 0

    lax.fori_loop(0, IW // LANES, fill_ones, 0)
    _zero_vec_ref(zbuf_v, ROWS_PER_TILE // LANES)
    pltpu.sync_copy(zbuf_v, accd_sh.at[pl.ds(s * ROWS_PER_TILE, ROWS_PER_TILE)])
    plsc.subcore_barrier()

    def win(j, _):
        pltpu.sync_copy(ones_v, accd_sh.at[didx_v.at[j]], add=True)
        return 0

    lax.fori_loop(0, NROW, win, 0)
    plsc.subcore_barrier()

    pltpu.sync_copy(
        accd_sh.at[pl.ds(s * ROWS_PER_TILE, ROWS_PER_TILE)],
        out_hbm.at[c].at[pl.ds(s * ROWS_PER_TILE, ROWS_PER_TILE)],
    )


def _sc_degree(didx_r):
    k = pl.kernel(
        _deg_body,
        out_type=jax.ShapeDtypeStruct((NC, N_PAD), _f32),
        mesh=_sc_mesh(),
        scratch_types=[
            pltpu.VMEM((NROW, IW), _i32),
            pltpu.VMEM((IW,), _f32),
            pltpu.VMEM((ROWS_PER_TILE,), _f32),
            pltpu.VMEM_SHARED((N_PAD,), _f32),
            pltpu.SemaphoreType.DMA,
        ],
        name="gcn_degree_sc",
    )
    return k(didx_r)


def _scat_body(hws_hbm, sidx_hbm, didx_hbm, out_hbm,
               sidx_v, didx_v, dsm_v, rows_v, acc_sh, sem0, sem1, sem2, sem3):
    c = lax.axis_index("c")
    s = lax.axis_index("s")
    wid = c * NS + s

    zrow = rows_v.at[0]

    def zrow_fill(i, _):
        zrow[i // (HID // LANES),
             pl.ds((i % (HID // LANES)) * LANES, LANES)] = jnp.zeros((LANES,), _f32)
        return 0

    lax.fori_loop(0, KW * HID // LANES, zrow_fill, 0)

    def zcopy(i, _):
        pltpu.sync_copy(zrow, acc_sh.at[pl.ds(s * ROWS_PER_TILE + i * KW, KW)])
        return 0

    lax.fori_loop(0, ROWS_PER_TILE // KW, zcopy, 0)
    plsc.subcore_barrier()

    sems = (sem0, sem1, sem2, sem3)
    crows = CH // 2

    def _sidx_win(j):
        return sidx_v.at[j // 2].at[pl.ds((j % 2) * KW, KW)]

    def chunk(ch, _):
        pltpu.sync_copy(sidx_hbm.at[wid].at[pl.ds(ch * crows, crows)], sidx_v)
        pltpu.sync_copy(didx_hbm.at[wid].at[pl.ds(ch * crows, crows)], didx_v)

        for b in range(SLOTS):
            pltpu.async_copy(hws_hbm.at[_sidx_win(b)], rows_v.at[b], sems[b])

        def win(w, _):
            for b in range(SLOTS):
                j = w * SLOTS + b
                buf = rows_v.at[b]
                pltpu.make_async_copy(hws_hbm.at[_sidx_win(j)], buf, sems[b]).wait()
                for k in range(KW // LANES):
                    dsm_v[b, pl.ds(k * LANES, LANES)] = (
                        didx_v[j // 2, pl.ds((j % 2) * KW + k * LANES, LANES)])
                pltpu.sync_copy(buf, acc_sh.at[dsm_v.at[b]], add=True)

                @pl.when(j + SLOTS < CH)
                def _():
                    pltpu.async_copy(hws_hbm.at[_sidx_win(j + SLOTS)],
                                     buf, sems[b])
            return 0

        lax.fori_loop(0, CH // SLOTS, win, 0)
        return 0

    lax.fori_loop(0, NCH, chunk, 0)
    plsc.subcore_barrier()

    pltpu.sync_copy(
        acc_sh.at[pl.ds(s * ROWS_PER_TILE, ROWS_PER_TILE)],
        out_hbm.at[c].at[pl.ds(s * ROWS_PER_TILE, ROWS_PER_TILE)],
    )


def _sc_gather_scatter(hws, sidx_r, didx_r):
    k = pl.kernel(
        _scat_body,
        out_type=jax.ShapeDtypeStruct((NC, N_PAD, HID), _f32),
        mesh=_sc_mesh(),
        scratch_types=[
            pltpu.VMEM((CH // 2, IW), _i32),
            pltpu.VMEM((CH // 2, IW), _i32),
            pltpu.VMEM((SLOTS, KW), _i32),
            pltpu.VMEM((SLOTS, KW, HID), _f32),
            pltpu.VMEM_SHARED((N_PAD, HID), _f32),
            pltpu.SemaphoreType.DMA,
            pltpu.SemaphoreType.DMA,
            pltpu.SemaphoreType.DMA,
            pltpu.SemaphoreType.DMA,
        ],
        name="gcn_gather_scatter_sc",
    )
    return k(hws, sidx_r, didx_r)


def _dinv_from_pdeg(pdeg_ref):
    deg = pdeg_ref[:, 0:1] + pdeg_ref[:, 1:2] + 1.0
    return 1.0 / jnp.sqrt(deg)


def _encode_kernel(x_ref, win_ref, bin_ref, w0_ref, pdeg_ref, out_ref):
    h0 = jnp.dot(x_ref[...], win_ref[...], preferred_element_type=_f32) + bin_ref[...]
    hw0 = jnp.dot(h0, w0_ref[...], preferred_element_type=_f32)
    out_ref[...] = _dinv_from_pdeg(pdeg_ref) * hw0


def _tc_encode(x_pad, w_in, b_in, w0, pdeg_t):
    return pl.pallas_call(
        _encode_kernel,
        grid=(N_BLKS,),
        out_shape=jax.ShapeDtypeStruct((N_PAD, HID), _f32),
        in_specs=[
            pl.BlockSpec((ROW_BLK, D_IN), lambda i: (i, 0)),
            pl.BlockSpec((D_IN, HID), lambda i: (0, 0)),
            pl.BlockSpec((HID,), lambda i: (0,)),
            pl.BlockSpec((HID, HID), lambda i: (0, 0)),
            pl.BlockSpec((ROW_BLK, NC), lambda i: (i, 0)),
        ],
        out_specs=pl.BlockSpec((ROW_BLK, HID), lambda i: (i, 0)),
    )(x_pad, w_in, b_in, w0, pdeg_t)


def _mid_kernel(acc_ref, hws_ref, pdeg_ref, b_ref, w_ref, out_ref):
    dinv = _dinv_from_pdeg(pdeg_ref)
    t = acc_ref[0] + acc_ref[1] + hws_ref[...]
    h = jnp.maximum(dinv * t + b_ref[...], 0.0)
    out_ref[...] = dinv * jnp.dot(h, w_ref[...], preferred_element_type=_f32)


def _tc_mid(acc, hws, pdeg_t, b, w):
    return pl.pallas_call(
        _mid_kernel,
        grid=(N_BLKS,),
        out_shape=jax.ShapeDtypeStruct((N_PAD, HID), _f32),
        in_specs=[
            pl.BlockSpec((NC, ROW_BLK, HID), lambda i: (0, i, 0)),
            pl.BlockSpec((ROW_BLK, HID), lambda i: (i, 0)),
            pl.BlockSpec((ROW_BLK, NC), lambda i: (i, 0)),
            pl.BlockSpec((HID,), lambda i: (0,)),
            pl.BlockSpec((HID, HID), lambda i: (0, 0)),
        ],
        out_specs=pl.BlockSpec((ROW_BLK, HID), lambda i: (i, 0)),
    )(acc, hws, pdeg_t, b, w)


def _final_kernel(acc_ref, hws_ref, pdeg_ref, b_ref, wcls_ref, bcls_ref, bid_ref,
                  out_ref, sums_acc, cnts_acc):
    i = pl.program_id(0)

    @pl.when(i == 0)
    def _():
        sums_acc[...] = jnp.zeros_like(sums_acc)
        cnts_acc[...] = jnp.zeros_like(cnts_acc)

    t = acc_ref[0] + acc_ref[1] + hws_ref[...]
    h = _dinv_from_pdeg(pdeg_ref) * t + b_ref[...]
    y = jnp.dot(h, wcls_ref[...], preferred_element_type=_f32) + bcls_ref[...]

    bid = bid_ref[0]
    iota = lax.broadcasted_iota(_i32, (N_GROUPS, ROW_BLK), 0)
    mask = (bid == iota).astype(_f32)
    sums_acc[...] += jnp.dot(mask, y, preferred_element_type=_f32)
    cnts_acc[...] += jnp.broadcast_to(
        jnp.sum(mask, axis=1, keepdims=True), (N_GROUPS, D_OUT)
    )

    @pl.when(i == N_BLKS - 1)
    def _():
        out_ref[...] = sums_acc[...] / jnp.maximum(cnts_acc[...], 1.0)


def _tc_final(acc, hws, pdeg_t, b2, w_cls, b_cls, bid3):
    return pl.pallas_call(
        _final_kernel,
        grid=(N_BLKS,),
        out_shape=jax.ShapeDtypeStruct((N_GROUPS, D_OUT), _f32),
        in_specs=[
            pl.BlockSpec((NC, ROW_BLK, HID), lambda i: (0, i, 0)),
            pl.BlockSpec((ROW_BLK, HID), lambda i: (i, 0)),
            pl.BlockSpec((ROW_BLK, NC), lambda i: (i, 0)),
            pl.BlockSpec((HID,), lambda i: (0,)),
            pl.BlockSpec((HID, D_OUT), lambda i: (0, 0)),
            pl.BlockSpec((D_OUT,), lambda i: (0,)),
            pl.BlockSpec((1, 1, ROW_BLK), lambda i: (i, 0, 0)),
        ],
        out_specs=pl.BlockSpec((N_GROUPS, D_OUT), lambda i: (0, 0)),
        scratch_shapes=[
            pltpu.VMEM((N_GROUPS, D_OUT), _f32),
            pltpu.VMEM((N_GROUPS, D_OUT), _f32),
        ],
    )(acc, hws, pdeg_t, b2, w_cls, b_cls, bid3)


def kernel(x, edge_index, batch_ids, W_in, b_in, W0, b0, W1, b1, W2, b2,
           W_cls, b_cls):
    n_extra = E_PAD - N_EDGES
    sidx_pad = jnp.concatenate(
        [edge_index[0], (jnp.arange(n_extra, dtype=_i32) % N_NODES)])
    didx_pad = jnp.concatenate(
        [edge_index[1], N_NODES + (jnp.arange(n_extra, dtype=_i32) % N_DUMP)])
    sidx_r = sidx_pad.reshape(NW, NROW, IW)
    didx_r = didx_pad.reshape(NW, NROW, IW)

    x_pad = jnp.pad(x, ((0, N_PAD - N_NODES), (0, 0)))
    bid3 = jnp.pad(batch_ids, (0, N_PAD - N_NODES),
                   constant_values=N_GROUPS).reshape(N_BLKS, 1, ROW_BLK)

    pdeg_t = _sc_degree(didx_r).T

    hws0 = _tc_encode(x_pad, W_in, b_in, W0, pdeg_t)
    acc0 = _sc_gather_scatter(hws0, sidx_r, didx_r)
    hws1 = _tc_mid(acc0, hws0, pdeg_t, b0, W1)
    acc1 = _sc_gather_scatter(hws1, sidx_r, didx_r)
    hws2 = _tc_mid(acc1, hws1, pdeg_t, b1, W2)
    acc2 = _sc_gather_scatter(hws2, sidx_r, didx_r)
    return _tc_final(acc2, hws2, pdeg_t, b2, W_cls, b_cls, bid3)


# speedup vs baseline: 1.1047x; 1.0017x over previous
import functools

import jax
import jax.numpy as jnp
from jax import lax
from jax.experimental import pallas as pl
from jax.experimental.pallas import tpu as pltpu
from jax.experimental.pallas import tpu_sc as plsc

N_NODES = 10000
N_EDGES = 320000
D_IN = 128
HID = 128
D_OUT = 64
N_GROUPS = 16

NC = 2
NS = 16
NW = NC * NS
LANES = 16

N_PAD = 10240
ROWS_PER_TILE = N_PAD // NS
E_PAD = 327680
EPW = E_PAD // NW
KW = 64
NWIN = EPW // KW
CH = 32
NCH = NWIN // CH
SLOTS = 4
IW = 128
NROW = EPW // IW
N_DUMP = N_PAD - N_NODES

ROW_BLK = 1000
N_BLKS = N_NODES // ROW_BLK

_f32 = jnp.float32
_i32 = jnp.int32


def _sc_mesh():
    return plsc.VectorSubcoreMesh(
        core_axis_name="c", subcore_axis_name="s", num_cores=NC, num_subcores=NS
    )


def _zero_vec_ref(ref, nvecs):
    zeros16 = jnp.zeros((LANES,), _f32)

    def body(i, _):
        ref[pl.ds(i * LANES, LANES)] = zeros16
        return 0

    lax.fori_loop(0, nvecs, body, 0)


def _deg_body(didx_hbm, out_hbm, didx_v, ones_v, zbuf_v, accd_sh, sem):
    del sem
    c = lax.axis_index("c")
    s = lax.axis_index("s")
    wid = c * NS + s

    pltpu.sync_copy(didx_hbm.at[wid], didx_v)

    ones16 = jnp.ones((LANES,), _f32)

    def fill_ones(i, _):
        ones_v[pl.ds(i * LANES, LANES)] = ones16
        return 0

    lax.fori_loop(0, IW // LANES, fill_ones, 0)
    _zero_vec_ref(zbuf_v, ROWS_PER_TILE // LANES)
    pltpu.sync_copy(zbuf_v, accd_sh.at[pl.ds(s * ROWS_PER_TILE, ROWS_PER_TILE)])
    plsc.subcore_barrier()

    def win(j, _):
        pltpu.sync_copy(ones_v, accd_sh.at[didx_v.at[j]], add=True)
        return 0

    lax.fori_loop(0, NROW, win, 0)
    plsc.subcore_barrier()

    pltpu.sync_copy(
        accd_sh.at[pl.ds(s * ROWS_PER_TILE, ROWS_PER_TILE)],
        out_hbm.at[c].at[pl.ds(s * ROWS_PER_TILE, ROWS_PER_TILE)],
    )


def _sc_degree(didx_r):
    k = pl.kernel(
        _deg_body,
        out_type=jax.ShapeDtypeStruct((NC, N_PAD), _f32),
        mesh=_sc_mesh(),
        scratch_types=[
            pltpu.VMEM((NROW, IW), _i32),
            pltpu.VMEM((IW,), _f32),
            pltpu.VMEM((ROWS_PER_TILE,), _f32),
            pltpu.VMEM_SHARED((N_PAD,), _f32),
            pltpu.SemaphoreType.DMA,
        ],
        name="gcn_degree_sc",
    )
    return k(didx_r)


def _scat_body(hws_hbm, sidx_hbm, didx_hbm, out_hbm,
               sidx_v, didx_v, dsm_v, rows_v, acc_sh, sem0, sem1, sem2, sem3):
    c = lax.axis_index("c")
    s = lax.axis_index("s")
    wid = c * NS + s

    zrow = rows_v.at[0]

    def zrow_fill(i, _):
        zrow[i // (HID // LANES),
             pl.ds((i % (HID // LANES)) * LANES, LANES)] = jnp.zeros((LANES,), _f32)
        return 0

    lax.fori_loop(0, KW * HID // LANES, zrow_fill, 0)

    def zcopy(i, _):
        pltpu.sync_copy(zrow, acc_sh.at[pl.ds(s * ROWS_PER_TILE + i * KW, KW)])
        return 0

    lax.fori_loop(0, ROWS_PER_TILE // KW, zcopy, 0)
    plsc.subcore_barrier()

    sems = (sem0, sem1, sem2, sem3)
    crows = CH // 2

    def _sidx_win(j):
        return sidx_v.at[j // 2].at[pl.ds((j % 2) * KW, KW)]

    def chunk(ch, _):
        pltpu.sync_copy(sidx_hbm.at[wid].at[pl.ds(ch * crows, crows)], sidx_v)
        pltpu.sync_copy(didx_hbm.at[wid].at[pl.ds(ch * crows, crows)], didx_v)

        for b in range(SLOTS):
            pltpu.async_copy(hws_hbm.at[_sidx_win(b)], rows_v.at[b], sems[b])

        def win(w, _):
            for b in range(SLOTS):
                j = w * SLOTS + b
                buf = rows_v.at[b]
                pltpu.make_async_copy(hws_hbm.at[_sidx_win(j)], buf, sems[b]).wait()
                for k in range(KW // LANES):
                    dsm_v[b, pl.ds(k * LANES, LANES)] = (
                        didx_v[j // 2, pl.ds((j % 2) * KW + k * LANES, LANES)])
                pltpu.sync_copy(buf, acc_sh.at[dsm_v.at[b]], add=True)

                @pl.when(j + SLOTS < CH)
                def _():
                    pltpu.async_copy(hws_hbm.at[_sidx_win(j + SLOTS)],
                                     buf, sems[b])
            return 0

        lax.fori_loop(0, CH // SLOTS, win, 0)
        return 0

    lax.fori_loop(0, NCH, chunk, 0)
    plsc.subcore_barrier()

    pltpu.sync_copy(
        acc_sh.at[pl.ds(s * ROWS_PER_TILE, ROWS_PER_TILE)],
        out_hbm.at[c].at[pl.ds(s * ROWS_PER_TILE, ROWS_PER_TILE)],
    )


def _sc_gather_scatter(hws, sidx_r, didx_r):
    k = pl.kernel(
        _scat_body,
        out_type=jax.ShapeDtypeStruct((NC, N_PAD, HID), _f32),
        mesh=_sc_mesh(),
        scratch_types=[
            pltpu.VMEM((CH // 2, IW), _i32),
            pltpu.VMEM((CH // 2, IW), _i32),
            pltpu.VMEM((SLOTS, KW), _i32),
            pltpu.VMEM((SLOTS, KW, HID), _f32),
            pltpu.VMEM_SHARED((N_PAD, HID), _f32),
            pltpu.SemaphoreType.DMA,
            pltpu.SemaphoreType.DMA,
            pltpu.SemaphoreType.DMA,
            pltpu.SemaphoreType.DMA,
        ],
        name="gcn_gather_scatter_sc",
    )
    return k(hws, sidx_r, didx_r)


def _dinv_from_pdeg(pdeg_ref):
    deg = pdeg_ref[:, 0:1] + pdeg_ref[:, 1:2] + 1.0
    return 1.0 / jnp.sqrt(deg)


def _encode_kernel(x_ref, win_ref, bin_ref, w0_ref, pdeg_ref, out_ref):
    h0 = jnp.dot(x_ref[...], win_ref[...], preferred_element_type=_f32) + bin_ref[...]
    hw0 = jnp.dot(h0, w0_ref[...], preferred_element_type=_f32)
    out_ref[...] = _dinv_from_pdeg(pdeg_ref) * hw0


def _tc_encode(x, w_in, b_in, w0, pdeg_t):
    return pl.pallas_call(
        _encode_kernel,
        grid=(N_BLKS,),
        out_shape=jax.ShapeDtypeStruct((N_NODES, HID), _f32),
        in_specs=[
            pl.BlockSpec((ROW_BLK, D_IN), lambda i: (i, 0)),
            pl.BlockSpec((D_IN, HID), lambda i: (0, 0)),
            pl.BlockSpec((HID,), lambda i: (0,)),
            pl.BlockSpec((HID, HID), lambda i: (0, 0)),
            pl.BlockSpec((ROW_BLK, NC), lambda i: (i, 0)),
        ],
        out_specs=pl.BlockSpec((ROW_BLK, HID), lambda i: (i, 0)),
    )(x, w_in, b_in, w0, pdeg_t)


def _mid_kernel(acc_ref, hws_ref, pdeg_ref, b_ref, w_ref, out_ref):
    dinv = _dinv_from_pdeg(pdeg_ref)
    t = acc_ref[0] + acc_ref[1] + hws_ref[...]
    h = jnp.maximum(dinv * t + b_ref[...], 0.0)
    out_ref[...] = dinv * jnp.dot(h, w_ref[...], preferred_element_type=_f32)


def _tc_mid(acc, hws, pdeg_t, b, w):
    return pl.pallas_call(
        _mid_kernel,
        grid=(N_BLKS,),
        out_shape=jax.ShapeDtypeStruct((N_NODES, HID), _f32),
        in_specs=[
            pl.BlockSpec((NC, ROW_BLK, HID), lambda i: (0, i, 0)),
            pl.BlockSpec((ROW_BLK, HID), lambda i: (i, 0)),
            pl.BlockSpec((ROW_BLK, NC), lambda i: (i, 0)),
            pl.BlockSpec((HID,), lambda i: (0,)),
            pl.BlockSpec((HID, HID), lambda i: (0, 0)),
        ],
        out_specs=pl.BlockSpec((ROW_BLK, HID), lambda i: (i, 0)),
    )(acc, hws, pdeg_t, b, w)


def _final_kernel(acc_ref, hws_ref, pdeg_ref, b_ref, wcls_ref, bcls_ref, bid_ref,
                  out_ref, sums_acc, cnts_acc):
    i = pl.program_id(0)

    @pl.when(i == 0)
    def _():
        sums_acc[...] = jnp.zeros_like(sums_acc)
        cnts_acc[...] = jnp.zeros_like(cnts_acc)

    t = acc_ref[0] + acc_ref[1] + hws_ref[...]
    h = _dinv_from_pdeg(pdeg_ref) * t + b_ref[...]
    y = jnp.dot(h, wcls_ref[...], preferred_element_type=_f32) + bcls_ref[...]

    bid = bid_ref[0]
    iota = lax.broadcasted_iota(_i32, (N_GROUPS, ROW_BLK), 0)
    mask = (bid == iota).astype(_f32)
    sums_acc[...] += jnp.dot(mask, y, preferred_element_type=_f32)
    cnts_acc[...] += jnp.broadcast_to(
        jnp.sum(mask, axis=1, keepdims=True), (N_GROUPS, D_OUT)
    )

    @pl.when(i == N_BLKS - 1)
    def _():
        out_ref[...] = sums_acc[...] / jnp.maximum(cnts_acc[...], 1.0)


def _tc_final(acc, hws, pdeg_t, b2, w_cls, b_cls, bid3):
    return pl.pallas_call(
        _final_kernel,
        grid=(N_BLKS,),
        out_shape=jax.ShapeDtypeStruct((N_GROUPS, D_OUT), _f32),
        in_specs=[
            pl.BlockSpec((NC, ROW_BLK, HID), lambda i: (0, i, 0)),
            pl.BlockSpec((ROW_BLK, HID), lambda i: (i, 0)),
            pl.BlockSpec((ROW_BLK, NC), lambda i: (i, 0)),
            pl.BlockSpec((HID,), lambda i: (0,)),
            pl.BlockSpec((HID, D_OUT), lambda i: (0, 0)),
            pl.BlockSpec((D_OUT,), lambda i: (0,)),
            pl.BlockSpec((1, 1, ROW_BLK), lambda i: (i, 0, 0)),
        ],
        out_specs=pl.BlockSpec((N_GROUPS, D_OUT), lambda i: (0, 0)),
        scratch_shapes=[
            pltpu.VMEM((N_GROUPS, D_OUT), _f32),
            pltpu.VMEM((N_GROUPS, D_OUT), _f32),
        ],
    )(acc, hws, pdeg_t, b2, w_cls, b_cls, bid3)


def kernel(x, edge_index, batch_ids, W_in, b_in, W0, b0, W1, b1, W2, b2,
           W_cls, b_cls):
    n_extra = E_PAD - N_EDGES
    sidx_pad = jnp.concatenate(
        [edge_index[0], (jnp.arange(n_extra, dtype=_i32) % N_NODES)])
    didx_pad = jnp.concatenate(
        [edge_index[1], N_NODES + (jnp.arange(n_extra, dtype=_i32) % N_DUMP)])
    sidx_r = sidx_pad.reshape(NW, NROW, IW)
    didx_r = didx_pad.reshape(NW, NROW, IW)

    bid3 = batch_ids.reshape(N_BLKS, 1, ROW_BLK)

    pdeg_t = _sc_degree(didx_r).T

    hws0 = _tc_encode(x, W_in, b_in, W0, pdeg_t)
    acc0 = _sc_gather_scatter(hws0, sidx_r, didx_r)
    hws1 = _tc_mid(acc0, hws0, pdeg_t, b0, W1)
    acc1 = _sc_gather_scatter(hws1, sidx_r, didx_r)
    hws2 = _tc_mid(acc1, hws1, pdeg_t, b1, W2)
    acc2 = _sc_gather_scatter(hws2, sidx_r, didx_r)
    return _tc_final(acc2, hws2, pdeg_t, b2, W_cls, b_cls, bid3)


# speedup vs baseline: 1.1281x; 1.0211x over previous
import functools

import jax
import jax.numpy as jnp
from jax import lax
from jax.experimental import pallas as pl
from jax.experimental.pallas import tpu as pltpu
from jax.experimental.pallas import tpu_sc as plsc

N_NODES = 10000
N_EDGES = 320000
D_IN = 128
HID = 128
D_OUT = 64
N_GROUPS = 16

NC = 2
NS = 16
NW = NC * NS
LANES = 16

N_PAD = 10240
ROWS_PER_TILE = N_PAD // NS
E_PAD = 327680
EPW = E_PAD // NW
KW = 64
NWIN = EPW // KW
CH = 32
NCH = NWIN // CH
SLOTS = 4
IW = 128
NROW = EPW // IW
N_DUMP = N_PAD - N_NODES

ROW_BLK = 2000
N_BLKS = N_NODES // ROW_BLK

_f32 = jnp.float32
_i32 = jnp.int32


def _sc_mesh():
    return plsc.VectorSubcoreMesh(
        core_axis_name="c", subcore_axis_name="s", num_cores=NC, num_subcores=NS
    )


def _zero_vec_ref(ref, nvecs):
    zeros16 = jnp.zeros((LANES,), _f32)

    def body(i, _):
        ref[pl.ds(i * LANES, LANES)] = zeros16
        return 0

    lax.fori_loop(0, nvecs, body, 0)


def _deg_body(didx_hbm, out_hbm, didx_v, ones_v, zbuf_v, accd_sh, sem):
    del sem
    c = lax.axis_index("c")
    s = lax.axis_index("s")
    wid = c * NS + s

    pltpu.sync_copy(didx_hbm.at[wid], didx_v)

    ones16 = jnp.ones((LANES,), _f32)

    def fill_ones(i, _):
        ones_v[pl.ds(i * LANES, LANES)] = ones16
        return 0

    lax.fori_loop(0, IW // LANES, fill_ones, 0)
    _zero_vec_ref(zbuf_v, ROWS_PER_TILE // LANES)
    pltpu.sync_copy(zbuf_v, accd_sh.at[pl.ds(s * ROWS_PER_TILE, ROWS_PER_TILE)])
    plsc.subcore_barrier()

    def win(j, _):
        pltpu.sync_copy(ones_v, accd_sh.at[didx_v.at[j]], add=True)
        return 0

    lax.fori_loop(0, NROW, win, 0)
    plsc.subcore_barrier()

    pltpu.sync_copy(
        accd_sh.at[pl.ds(s * ROWS_PER_TILE, ROWS_PER_TILE)],
        out_hbm.at[c].at[pl.ds(s * ROWS_PER_TILE, ROWS_PER_TILE)],
    )


def _sc_degree(didx_r):
    k = pl.kernel(
        _deg_body,
        out_type=jax.ShapeDtypeStruct((NC, N_PAD), _f32),
        mesh=_sc_mesh(),
        scratch_types=[
            pltpu.VMEM((NROW, IW), _i32),
            pltpu.VMEM((IW,), _f32),
            pltpu.VMEM((ROWS_PER_TILE,), _f32),
            pltpu.VMEM_SHARED((N_PAD,), _f32),
            pltpu.SemaphoreType.DMA,
        ],
        name="gcn_degree_sc",
    )
    return k(didx_r)


def _scat_body(hws_hbm, sidx_hbm, didx_hbm, out_hbm,
               sidx_v, didx_v, dsm_v, rows_v, acc_sh, sem0, sem1, sem2, sem3):
    c = lax.axis_index("c")
    s = lax.axis_index("s")
    wid = c * NS + s

    zrow = rows_v.at[0]

    def zrow_fill(i, _):
        zrow[i // (HID // LANES),
             pl.ds((i % (HID // LANES)) * LANES, LANES)] = jnp.zeros((LANES,), _f32)
        return 0

    lax.fori_loop(0, KW * HID // LANES, zrow_fill, 0)

    def zcopy(i, _):
        pltpu.sync_copy(zrow, acc_sh.at[pl.ds(s * ROWS_PER_TILE + i * KW, KW)])
        return 0

    lax.fori_loop(0, ROWS_PER_TILE // KW, zcopy, 0)
    plsc.subcore_barrier()

    sems = (sem0, sem1, sem2, sem3)
    crows = CH // 2

    def _sidx_win(j):
        return sidx_v.at[j // 2].at[pl.ds((j % 2) * KW, KW)]

    def chunk(ch, _):
        pltpu.sync_copy(sidx_hbm.at[wid].at[pl.ds(ch * crows, crows)], sidx_v)
        pltpu.sync_copy(didx_hbm.at[wid].at[pl.ds(ch * crows, crows)], didx_v)

        for b in range(SLOTS):
            pltpu.async_copy(hws_hbm.at[_sidx_win(b)], rows_v.at[b], sems[b])

        def win(w, _):
            for b in range(SLOTS):
                j = w * SLOTS + b
                buf = rows_v.at[b]
                pltpu.make_async_copy(hws_hbm.at[_sidx_win(j)], buf, sems[b]).wait()
                for k in range(KW // LANES):
                    dsm_v[b, pl.ds(k * LANES, LANES)] = (
                        didx_v[j // 2, pl.ds((j % 2) * KW + k * LANES, LANES)])
                pltpu.sync_copy(buf, acc_sh.at[dsm_v.at[b]], add=True)

                @pl.when(j + SLOTS < CH)
                def _():
                    pltpu.async_copy(hws_hbm.at[_sidx_win(j + SLOTS)],
                                     buf, sems[b])
            return 0

        lax.fori_loop(0, CH // SLOTS, win, 0)
        return 0

    lax.fori_loop(0, NCH, chunk, 0)
    plsc.subcore_barrier()

    pltpu.sync_copy(
        acc_sh.at[pl.ds(s * ROWS_PER_TILE, ROWS_PER_TILE)],
        out_hbm.at[c].at[pl.ds(s * ROWS_PER_TILE, ROWS_PER_TILE)],
    )


def _sc_gather_scatter(hws, sidx_r, didx_r):
    k = pl.kernel(
        _scat_body,
        out_type=jax.ShapeDtypeStruct((NC, N_PAD, HID), _f32),
        mesh=_sc_mesh(),
        scratch_types=[
            pltpu.VMEM((CH // 2, IW), _i32),
            pltpu.VMEM((CH // 2, IW), _i32),
            pltpu.VMEM((SLOTS, KW), _i32),
            pltpu.VMEM((SLOTS, KW, HID), _f32),
            pltpu.VMEM_SHARED((N_PAD, HID), _f32),
            pltpu.SemaphoreType.DMA,
            pltpu.SemaphoreType.DMA,
            pltpu.SemaphoreType.DMA,
            pltpu.SemaphoreType.DMA,
        ],
        name="gcn_gather_scatter_sc",
    )
    return k(hws, sidx_r, didx_r)


def _dinv_from_pdeg(pdeg_ref):
    deg = pdeg_ref[:, 0:1] + pdeg_ref[:, 1:2] + 1.0
    return 1.0 / jnp.sqrt(deg)


def _encode_kernel(x_ref, win_ref, bin_ref, w0_ref, pdeg_ref, out_ref):
    h0 = jnp.dot(x_ref[...], win_ref[...], preferred_element_type=_f32) + bin_ref[...]
    hw0 = jnp.dot(h0, w0_ref[...], preferred_element_type=_f32)
    out_ref[...] = _dinv_from_pdeg(pdeg_ref) * hw0


def _tc_encode(x, w_in, b_in, w0, pdeg_t):
    return pl.pallas_call(
        _encode_kernel,
        grid=(N_BLKS,),
        out_shape=jax.ShapeDtypeStruct((N_NODES, HID), _f32),
        in_specs=[
            pl.BlockSpec((ROW_BLK, D_IN), lambda i: (i, 0)),
            pl.BlockSpec((D_IN, HID), lambda i: (0, 0)),
            pl.BlockSpec((HID,), lambda i: (0,)),
            pl.BlockSpec((HID, HID), lambda i: (0, 0)),
            pl.BlockSpec((ROW_BLK, NC), lambda i: (i, 0)),
        ],
        out_specs=pl.BlockSpec((ROW_BLK, HID), lambda i: (i, 0)),
    )(x, w_in, b_in, w0, pdeg_t)


def _mid_kernel(acc_ref, hws_ref, pdeg_ref, b_ref, w_ref, out_ref):
    dinv = _dinv_from_pdeg(pdeg_ref)
    t = acc_ref[0] + acc_ref[1] + hws_ref[...]
    h = jnp.maximum(dinv * t + b_ref[...], 0.0)
    out_ref[...] = dinv * jnp.dot(h, w_ref[...], preferred_element_type=_f32)


def _tc_mid(acc, hws, pdeg_t, b, w):
    return pl.pallas_call(
        _mid_kernel,
        grid=(N_BLKS,),
        out_shape=jax.ShapeDtypeStruct((N_NODES, HID), _f32),
        in_specs=[
            pl.BlockSpec((NC, ROW_BLK, HID), lambda i: (0, i, 0)),
            pl.BlockSpec((ROW_BLK, HID), lambda i: (i, 0)),
            pl.BlockSpec((ROW_BLK, NC), lambda i: (i, 0)),
            pl.BlockSpec((HID,), lambda i: (0,)),
            pl.BlockSpec((HID, HID), lambda i: (0, 0)),
        ],
        out_specs=pl.BlockSpec((ROW_BLK, HID), lambda i: (i, 0)),
    )(acc, hws, pdeg_t, b, w)


def _final_kernel(acc_ref, hws_ref, pdeg_ref, b_ref, wcls_ref, bcls_ref, bid_ref,
                  out_ref, sums_acc, cnts_acc):
    i = pl.program_id(0)

    @pl.when(i == 0)
    def _():
        sums_acc[...] = jnp.zeros_like(sums_acc)
        cnts_acc[...] = jnp.zeros_like(cnts_acc)

    t = acc_ref[0] + acc_ref[1] + hws_ref[...]
    h = _dinv_from_pdeg(pdeg_ref) * t + b_ref[...]
    y = jnp.dot(h, wcls_ref[...], preferred_element_type=_f32) + bcls_ref[...]

    bid = bid_ref[0]
    iota = lax.broadcasted_iota(_i32, (N_GROUPS, ROW_BLK), 0)
    mask = (bid == iota).astype(_f32)
    sums_acc[...] += jnp.dot(mask, y, preferred_element_type=_f32)
    cnts_acc[...] += jnp.broadcast_to(
        jnp.sum(mask, axis=1, keepdims=True), (N_GROUPS, D_OUT)
    )

    @pl.when(i == N_BLKS - 1)
    def _():
        out_ref[...] = sums_acc[...] / jnp.maximum(cnts_acc[...], 1.0)


def _tc_final(acc, hws, pdeg_t, b2, w_cls, b_cls, bid3):
    return pl.pallas_call(
        _final_kernel,
        grid=(N_BLKS,),
        out_shape=jax.ShapeDtypeStruct((N_GROUPS, D_OUT), _f32),
        in_specs=[
            pl.BlockSpec((NC, ROW_BLK, HID), lambda i: (0, i, 0)),
            pl.BlockSpec((ROW_BLK, HID), lambda i: (i, 0)),
            pl.BlockSpec((ROW_BLK, NC), lambda i: (i, 0)),
            pl.BlockSpec((HID,), lambda i: (0,)),
            pl.BlockSpec((HID, D_OUT), lambda i: (0, 0)),
            pl.BlockSpec((D_OUT,), lambda i: (0,)),
            pl.BlockSpec((1, 1, ROW_BLK), lambda i: (i, 0, 0)),
        ],
        out_specs=pl.BlockSpec((N_GROUPS, D_OUT), lambda i: (0, 0)),
        scratch_shapes=[
            pltpu.VMEM((N_GROUPS, D_OUT), _f32),
            pltpu.VMEM((N_GROUPS, D_OUT), _f32),
        ],
    )(acc, hws, pdeg_t, b2, w_cls, b_cls, bid3)


def kernel(x, edge_index, batch_ids, W_in, b_in, W0, b0, W1, b1, W2, b2,
           W_cls, b_cls):
    n_extra = E_PAD - N_EDGES
    sidx_pad = jnp.concatenate(
        [edge_index[0], (jnp.arange(n_extra, dtype=_i32) % N_NODES)])
    didx_pad = jnp.concatenate(
        [edge_index[1], N_NODES + (jnp.arange(n_extra, dtype=_i32) % N_DUMP)])
    sidx_r = sidx_pad.reshape(NW, NROW, IW)
    didx_r = didx_pad.reshape(NW, NROW, IW)

    bid3 = batch_ids.reshape(N_BLKS, 1, ROW_BLK)

    pdeg_t = _sc_degree(didx_r).T

    hws0 = _tc_encode(x, W_in, b_in, W0, pdeg_t)
    acc0 = _sc_gather_scatter(hws0, sidx_r, didx_r)
    hws1 = _tc_mid(acc0, hws0, pdeg_t, b0, W1)
    acc1 = _sc_gather_scatter(hws1, sidx_r, didx_r)
    hws2 = _tc_mid(acc1, hws1, pdeg_t, b1, W2)
    acc2 = _sc_gather_scatter(hws2, sidx_r, didx_r)
    return _tc_final(acc2, hws2, pdeg_t, b2, W_cls, b_cls, bid3)


# speedup vs baseline: 1.1485x; 1.0181x over previous
import functools

import jax
import jax.numpy as jnp
from jax import lax
from jax.experimental import pallas as pl
from jax.experimental.pallas import tpu as pltpu
from jax.experimental.pallas import tpu_sc as plsc

N_NODES = 10000
N_EDGES = 320000
D_IN = 128
HID = 128
D_OUT = 64
N_GROUPS = 16

NC = 2
NS = 16
NW = NC * NS
LANES = 16

N_PAD = 10240
ROWS_PER_TILE = N_PAD // NS
E_PAD = 327680
EPW = E_PAD // NW
KW = 64
NWIN = EPW // KW
CH = 32
NCH = NWIN // CH
SLOTS = 4
IW = 128
NROW = EPW // IW
N_DUMP = N_PAD - N_NODES

ROW_BLK = 2000
N_BLKS = N_NODES // ROW_BLK

_f32 = jnp.float32
_i32 = jnp.int32


def _sc_mesh():
    return plsc.VectorSubcoreMesh(
        core_axis_name="c", subcore_axis_name="s", num_cores=NC, num_subcores=NS
    )


def _zero_vec_ref(ref, nvecs):
    zeros16 = jnp.zeros((LANES,), _f32)

    def body(i, _):
        ref[pl.ds(i * LANES, LANES)] = zeros16
        return 0

    lax.fori_loop(0, nvecs, body, 0)


def _deg_body(eidx_hbm, out_hbm, didx_v, dsm_v, ones_v, zbuf_v, accd_sh, sem):
    del sem
    c = lax.axis_index("c")
    s = lax.axis_index("s")
    wid = c * NS + s

    pltpu.sync_copy(eidx_hbm.at[1].at[pl.ds(wid * EPW, EPW)], didx_v)

    ones16 = jnp.ones((LANES,), _f32)

    def fill_ones(i, _):
        ones_v[pl.ds(i * LANES, LANES)] = ones16
        return 0

    lax.fori_loop(0, IW // LANES, fill_ones, 0)
    _zero_vec_ref(zbuf_v, ROWS_PER_TILE // LANES)
    pltpu.sync_copy(zbuf_v, accd_sh.at[pl.ds(s * ROWS_PER_TILE, ROWS_PER_TILE)])
    plsc.subcore_barrier()

    def win(j, _):
        for k in range(IW // LANES):
            dsm_v[0, pl.ds(k * LANES, LANES)] = (
                didx_v[pl.ds(j * IW + k * LANES, LANES)])
        pltpu.sync_copy(ones_v, accd_sh.at[dsm_v.at[0]], add=True)
        return 0

    lax.fori_loop(0, NROW, win, 0)
    plsc.subcore_barrier()

    pltpu.sync_copy(
        accd_sh.at[pl.ds(s * ROWS_PER_TILE, ROWS_PER_TILE)],
        out_hbm.at[c].at[pl.ds(s * ROWS_PER_TILE, ROWS_PER_TILE)],
    )


def _sc_degree(eidx):
    k = pl.kernel(
        _deg_body,
        out_type=jax.ShapeDtypeStruct((NC, N_PAD), _f32),
        mesh=_sc_mesh(),
        scratch_types=[
            pltpu.VMEM((EPW,), _i32),
            pltpu.VMEM((1, IW), _i32),
            pltpu.VMEM((IW,), _f32),
            pltpu.VMEM((ROWS_PER_TILE,), _f32),
            pltpu.VMEM_SHARED((N_PAD,), _f32),
            pltpu.SemaphoreType.DMA,
        ],
        name="gcn_degree_sc",
    )
    return k(eidx)


def _scat_body(hws_hbm, eidx_hbm, out_hbm,
               sidx_v, didx_v, dsm_v, rows_v, acc_sh, sem0, sem1, sem2, sem3):
    c = lax.axis_index("c")
    s = lax.axis_index("s")
    wid = c * NS + s

    zrow = rows_v.at[0]

    def zrow_fill(i, _):
        zrow[i // (HID // LANES),
             pl.ds((i % (HID // LANES)) * LANES, LANES)] = jnp.zeros((LANES,), _f32)
        return 0

    lax.fori_loop(0, KW * HID // LANES, zrow_fill, 0)

    def zcopy(i, _):
        pltpu.sync_copy(zrow, acc_sh.at[pl.ds(s * ROWS_PER_TILE + i * KW, KW)])
        return 0

    lax.fori_loop(0, ROWS_PER_TILE // KW, zcopy, 0)
    plsc.subcore_barrier()

    sems = (sem0, sem1, sem2, sem3)
    cw = CH * KW

    def _sidx_win(j):
        return sidx_v.at[pl.ds(j * KW, KW)]

    def chunk(ch, _):
        base = wid * EPW + ch * cw
        pltpu.sync_copy(eidx_hbm.at[0].at[pl.ds(base, cw)], sidx_v)
        pltpu.sync_copy(eidx_hbm.at[1].at[pl.ds(base, cw)], didx_v)

        for b in range(SLOTS):
            pltpu.async_copy(hws_hbm.at[_sidx_win(b)], rows_v.at[b], sems[b])

        def win(w, _):
            for b in range(SLOTS):
                j = w * SLOTS + b
                buf = rows_v.at[b]
                pltpu.make_async_copy(hws_hbm.at[_sidx_win(j)], buf, sems[b]).wait()
                for k in range(KW // LANES):
                    dsm_v[b, pl.ds(k * LANES, LANES)] = (
                        didx_v[pl.ds(j * KW + k * LANES, LANES)])
                pltpu.sync_copy(buf, acc_sh.at[dsm_v.at[b]], add=True)

                @pl.when(j + SLOTS < CH)
                def _():
                    pltpu.async_copy(hws_hbm.at[_sidx_win(j + SLOTS)],
                                     buf, sems[b])
            return 0

        lax.fori_loop(0, CH // SLOTS, win, 0)
        return 0

    lax.fori_loop(0, NCH, chunk, 0)
    plsc.subcore_barrier()

    pltpu.sync_copy(
        acc_sh.at[pl.ds(s * ROWS_PER_TILE, ROWS_PER_TILE)],
        out_hbm.at[c].at[pl.ds(s * ROWS_PER_TILE, ROWS_PER_TILE)],
    )


def _sc_gather_scatter(hws, eidx):
    k = pl.kernel(
        _scat_body,
        out_type=jax.ShapeDtypeStruct((NC, N_PAD, HID), _f32),
        mesh=_sc_mesh(),
        scratch_types=[
            pltpu.VMEM((CH * KW,), _i32),
            pltpu.VMEM((CH * KW,), _i32),
            pltpu.VMEM((SLOTS, KW), _i32),
            pltpu.VMEM((SLOTS, KW, HID), _f32),
            pltpu.VMEM_SHARED((N_PAD, HID), _f32),
            pltpu.SemaphoreType.DMA,
            pltpu.SemaphoreType.DMA,
            pltpu.SemaphoreType.DMA,
            pltpu.SemaphoreType.DMA,
        ],
        name="gcn_gather_scatter_sc",
    )
    return k(hws, eidx)


def _dinv_from_pdeg(pdeg_ref):
    deg = pdeg_ref[:, 0:1] + pdeg_ref[:, 1:2] + 1.0
    return 1.0 / jnp.sqrt(deg)


def _encode_kernel(x_ref, win_ref, bin_ref, w0_ref, pdeg_ref, out_ref):
    h0 = jnp.dot(x_ref[...], win_ref[...], preferred_element_type=_f32) + bin_ref[...]
    hw0 = jnp.dot(h0, w0_ref[...], preferred_element_type=_f32)
    out_ref[...] = _dinv_from_pdeg(pdeg_ref) * hw0


def _tc_encode(x, w_in, b_in, w0, pdeg_t):
    return pl.pallas_call(
        _encode_kernel,
        grid=(N_BLKS,),
        out_shape=jax.ShapeDtypeStruct((N_NODES, HID), _f32),
        in_specs=[
            pl.BlockSpec((ROW_BLK, D_IN), lambda i: (i, 0)),
            pl.BlockSpec((D_IN, HID), lambda i: (0, 0)),
            pl.BlockSpec((HID,), lambda i: (0,)),
            pl.BlockSpec((HID, HID), lambda i: (0, 0)),
            pl.BlockSpec((ROW_BLK, NC), lambda i: (i, 0)),
        ],
        out_specs=pl.BlockSpec((ROW_BLK, HID), lambda i: (i, 0)),
    )(x, w_in, b_in, w0, pdeg_t)


def _mid_kernel(acc_ref, hws_ref, pdeg_ref, b_ref, w_ref, out_ref):
    dinv = _dinv_from_pdeg(pdeg_ref)
    t = acc_ref[0] + acc_ref[1] + hws_ref[...]
    h = jnp.maximum(dinv * t + b_ref[...], 0.0)
    out_ref[...] = dinv * jnp.dot(h, w_ref[...], preferred_element_type=_f32)


def _tc_mid(acc, hws, pdeg_t, b, w):
    return pl.pallas_call(
        _mid_kernel,
        grid=(N_BLKS,),
        out_shape=jax.ShapeDtypeStruct((N_NODES, HID), _f32),
        in_specs=[
            pl.BlockSpec((NC, ROW_BLK, HID), lambda i: (0, i, 0)),
            pl.BlockSpec((ROW_BLK, HID), lambda i: (i, 0)),
            pl.BlockSpec((ROW_BLK, NC), lambda i: (i, 0)),
            pl.BlockSpec((HID,), lambda i: (0,)),
            pl.BlockSpec((HID, HID), lambda i: (0, 0)),
        ],
        out_specs=pl.BlockSpec((ROW_BLK, HID), lambda i: (i, 0)),
    )(acc, hws, pdeg_t, b, w)


def _final_kernel(acc_ref, hws_ref, pdeg_ref, b_ref, wcls_ref, bcls_ref, bid_ref,
                  out_ref, sums_acc, cnts_acc):
    i = pl.program_id(0)

    @pl.when(i == 0)
    def _():
        sums_acc[...] = jnp.zeros_like(sums_acc)
        cnts_acc[...] = jnp.zeros_like(cnts_acc)

    t = acc_ref[0] + acc_ref[1] + hws_ref[...]
    h = _dinv_from_pdeg(pdeg_ref) * t + b_ref[...]
    y = jnp.dot(h, wcls_ref[...], preferred_element_type=_f32) + bcls_ref[...]

    bid = bid_ref[0]
    iota = lax.broadcasted_iota(_i32, (N_GROUPS, ROW_BLK), 0)
    mask = (bid == iota).astype(_f32)
    sums_acc[...] += jnp.dot(mask, y, preferred_element_type=_f32)
    cnts_acc[...] += jnp.broadcast_to(
        jnp.sum(mask, axis=1, keepdims=True), (N_GROUPS, D_OUT)
    )

    @pl.when(i == N_BLKS - 1)
    def _():
        out_ref[...] = sums_acc[...] / jnp.maximum(cnts_acc[...], 1.0)


def _tc_final(acc, hws, pdeg_t, b2, w_cls, b_cls, bid3):
    return pl.pallas_call(
        _final_kernel,
        grid=(N_BLKS,),
        out_shape=jax.ShapeDtypeStruct((N_GROUPS, D_OUT), _f32),
        in_specs=[
            pl.BlockSpec((NC, ROW_BLK, HID), lambda i: (0, i, 0)),
            pl.BlockSpec((ROW_BLK, HID), lambda i: (i, 0)),
            pl.BlockSpec((ROW_BLK, NC), lambda i: (i, 0)),
            pl.BlockSpec((HID,), lambda i: (0,)),
            pl.BlockSpec((HID, D_OUT), lambda i: (0, 0)),
            pl.BlockSpec((D_OUT,), lambda i: (0,)),
            pl.BlockSpec((1, 1, ROW_BLK), lambda i: (i, 0, 0)),
        ],
        out_specs=pl.BlockSpec((N_GROUPS, D_OUT), lambda i: (0, 0)),
        scratch_shapes=[
            pltpu.VMEM((N_GROUPS, D_OUT), _f32),
            pltpu.VMEM((N_GROUPS, D_OUT), _f32),
        ],
    )(acc, hws, pdeg_t, b2, w_cls, b_cls, bid3)


def kernel(x, edge_index, batch_ids, W_in, b_in, W0, b0, W1, b1, W2, b2,
           W_cls, b_cls):
    n_extra = E_PAD - N_EDGES
    pad_edges = jnp.stack([
        jnp.arange(n_extra, dtype=_i32) % N_NODES,
        N_NODES + (jnp.arange(n_extra, dtype=_i32) % N_DUMP),
    ])
    eidx = jnp.concatenate([edge_index, pad_edges], axis=1)

    bid3 = batch_ids.reshape(N_BLKS, 1, ROW_BLK)

    pdeg_t = _sc_degree(eidx).T

    hws0 = _tc_encode(x, W_in, b_in, W0, pdeg_t)
    acc0 = _sc_gather_scatter(hws0, eidx)
    hws1 = _tc_mid(acc0, hws0, pdeg_t, b0, W1)
    acc1 = _sc_gather_scatter(hws1, eidx)
    hws2 = _tc_mid(acc1, hws1, pdeg_t, b1, W2)
    acc2 = _sc_gather_scatter(hws2, eidx)
    return _tc_final(acc2, hws2, pdeg_t, b2, W_cls, b_cls, bid3)


# speedup vs baseline: 1.1625x; 1.0121x over previous
import functools

import jax
import jax.numpy as jnp
from jax import lax
from jax.experimental import pallas as pl
from jax.experimental.pallas import tpu as pltpu
from jax.experimental.pallas import tpu_sc as plsc

N_NODES = 10000
N_EDGES = 320000
D_IN = 128
HID = 128
D_OUT = 64
N_GROUPS = 16

NC = 2
NS = 16
NW = NC * NS
LANES = 16

N_PAD = 10240
ROWS_PER_TILE = N_PAD // NS
E_PAD = 327680
EPW = E_PAD // NW
KW = 64
NWIN = EPW // KW
CH = 40
NCH = NWIN // CH
SLOTS = 5
IW = 128
NROW = EPW // IW
N_DUMP = N_PAD - N_NODES

ROW_BLK = 2000
N_BLKS = N_NODES // ROW_BLK

_f32 = jnp.float32
_i32 = jnp.int32


def _sc_mesh():
    return plsc.VectorSubcoreMesh(
        core_axis_name="c", subcore_axis_name="s", num_cores=NC, num_subcores=NS
    )


def _zero_vec_ref(ref, nvecs):
    zeros16 = jnp.zeros((LANES,), _f32)

    def body(i, _):
        ref[pl.ds(i * LANES, LANES)] = zeros16
        return 0

    lax.fori_loop(0, nvecs, body, 0)


def _deg_body(eidx_hbm, out_hbm, didx_v, dsm_v, ones_v, zbuf_v, accd_sh, sem):
    del sem
    c = lax.axis_index("c")
    s = lax.axis_index("s")
    wid = c * NS + s

    pltpu.sync_copy(eidx_hbm.at[1].at[pl.ds(wid * EPW, EPW)], didx_v)

    ones16 = jnp.ones((LANES,), _f32)

    def fill_ones(i, _):
        ones_v[pl.ds(i * LANES, LANES)] = ones16
        return 0

    lax.fori_loop(0, IW // LANES, fill_ones, 0)
    _zero_vec_ref(zbuf_v, ROWS_PER_TILE // LANES)
    pltpu.sync_copy(zbuf_v, accd_sh.at[pl.ds(s * ROWS_PER_TILE, ROWS_PER_TILE)])
    plsc.subcore_barrier()

    def win(j, _):
        for k in range(IW // LANES):
            dsm_v[0, pl.ds(k * LANES, LANES)] = (
                didx_v[pl.ds(j * IW + k * LANES, LANES)])
        pltpu.sync_copy(ones_v, accd_sh.at[dsm_v.at[0]], add=True)
        return 0

    lax.fori_loop(0, NROW, win, 0)
    plsc.subcore_barrier()

    pltpu.sync_copy(
        accd_sh.at[pl.ds(s * ROWS_PER_TILE, ROWS_PER_TILE)],
        out_hbm.at[c].at[pl.ds(s * ROWS_PER_TILE, ROWS_PER_TILE)],
    )


def _sc_degree(eidx):
    k = pl.kernel(
        _deg_body,
        out_type=jax.ShapeDtypeStruct((NC, N_PAD), _f32),
        mesh=_sc_mesh(),
        scratch_types=[
            pltpu.VMEM((EPW,), _i32),
            pltpu.VMEM((1, IW), _i32),
            pltpu.VMEM((IW,), _f32),
            pltpu.VMEM((ROWS_PER_TILE,), _f32),
            pltpu.VMEM_SHARED((N_PAD,), _f32),
            pltpu.SemaphoreType.DMA,
        ],
        name="gcn_degree_sc",
    )
    return k(eidx)


def _scat_body(hws_hbm, eidx_hbm, out_hbm, sidx_v, didx_v, dsm_v, rows_v,
               acc_sh, sem0, sem1, sem2, sem3, sem4):
    c = lax.axis_index("c")
    s = lax.axis_index("s")
    wid = c * NS + s

    zrow = rows_v.at[0]

    def zrow_fill(i, _):
        zrow[i // (HID // LANES),
             pl.ds((i % (HID // LANES)) * LANES, LANES)] = jnp.zeros((LANES,), _f32)
        return 0

    lax.fori_loop(0, KW * HID // LANES, zrow_fill, 0)

    def zcopy(i, _):
        pltpu.sync_copy(zrow, acc_sh.at[pl.ds(s * ROWS_PER_TILE + i * KW, KW)])
        return 0

    lax.fori_loop(0, ROWS_PER_TILE // KW, zcopy, 0)
    plsc.subcore_barrier()

    sems = (sem0, sem1, sem2, sem3, sem4)
    cw = CH * KW

    def _sidx_win(j):
        return sidx_v.at[pl.ds(j * KW, KW)]

    def chunk(ch, _):
        base = wid * EPW + ch * cw
        pltpu.sync_copy(eidx_hbm.at[0].at[pl.ds(base, cw)], sidx_v)
        pltpu.sync_copy(eidx_hbm.at[1].at[pl.ds(base, cw)], didx_v)

        for b in range(SLOTS):
            pltpu.async_copy(hws_hbm.at[_sidx_win(b)], rows_v.at[b], sems[b])

        def win(w, _):
            for b in range(SLOTS):
                j = w * SLOTS + b
                buf = rows_v.at[b]
                pltpu.make_async_copy(hws_hbm.at[_sidx_win(j)], buf, sems[b]).wait()
                for k in range(KW // LANES):
                    dsm_v[b, pl.ds(k * LANES, LANES)] = (
                        didx_v[pl.ds(j * KW + k * LANES, LANES)])
                pltpu.sync_copy(buf, acc_sh.at[dsm_v.at[b]], add=True)

                @pl.when(j + SLOTS < CH)
                def _():
                    pltpu.async_copy(hws_hbm.at[_sidx_win(j + SLOTS)],
                                     buf, sems[b])
            return 0

        lax.fori_loop(0, CH // SLOTS, win, 0)
        return 0

    lax.fori_loop(0, NCH, chunk, 0)
    plsc.subcore_barrier()

    pltpu.sync_copy(
        acc_sh.at[pl.ds(s * ROWS_PER_TILE, ROWS_PER_TILE)],
        out_hbm.at[c].at[pl.ds(s * ROWS_PER_TILE, ROWS_PER_TILE)],
    )


def _sc_gather_scatter(hws, eidx):
    k = pl.kernel(
        _scat_body,
        out_type=jax.ShapeDtypeStruct((NC, N_PAD, HID), _f32),
        mesh=_sc_mesh(),
        scratch_types=[
            pltpu.VMEM((CH * KW,), _i32),
            pltpu.VMEM((CH * KW,), _i32),
            pltpu.VMEM((SLOTS, KW), _i32),
            pltpu.VMEM((SLOTS, KW, HID), _f32),
            pltpu.VMEM_SHARED((N_PAD, HID), _f32),
            pltpu.SemaphoreType.DMA,
            pltpu.SemaphoreType.DMA,
            pltpu.SemaphoreType.DMA,
            pltpu.SemaphoreType.DMA,
            pltpu.SemaphoreType.DMA,
        ],
        name="gcn_gather_scatter_sc",
    )
    return k(hws, eidx)


def _dinv_from_pdeg(pdeg_ref):
    deg = pdeg_ref[:, 0:1] + pdeg_ref[:, 1:2] + 1.0
    return 1.0 / jnp.sqrt(deg)


def _encode_kernel(x_ref, win_ref, bin_ref, w0_ref, pdeg_ref, out_ref):
    h0 = jnp.dot(x_ref[...], win_ref[...], preferred_element_type=_f32) + bin_ref[...]
    hw0 = jnp.dot(h0, w0_ref[...], preferred_element_type=_f32)
    out_ref[...] = _dinv_from_pdeg(pdeg_ref) * hw0


def _tc_encode(x, w_in, b_in, w0, pdeg_t):
    return pl.pallas_call(
        _encode_kernel,
        grid=(N_BLKS,),
        out_shape=jax.ShapeDtypeStruct((N_NODES, HID), _f32),
        in_specs=[
            pl.BlockSpec((ROW_BLK, D_IN), lambda i: (i, 0)),
            pl.BlockSpec((D_IN, HID), lambda i: (0, 0)),
            pl.BlockSpec((HID,), lambda i: (0,)),
            pl.BlockSpec((HID, HID), lambda i: (0, 0)),
            pl.BlockSpec((ROW_BLK, NC), lambda i: (i, 0)),
        ],
        out_specs=pl.BlockSpec((ROW_BLK, HID), lambda i: (i, 0)),
    )(x, w_in, b_in, w0, pdeg_t)


def _mid_kernel(acc_ref, hws_ref, pdeg_ref, b_ref, w_ref, out_ref):
    dinv = _dinv_from_pdeg(pdeg_ref)
    t = acc_ref[0] + acc_ref[1] + hws_ref[...]
    h = jnp.maximum(dinv * t + b_ref[...], 0.0)
    out_ref[...] = dinv * jnp.dot(h, w_ref[...], preferred_element_type=_f32)


def _tc_mid(acc, hws, pdeg_t, b, w):
    return pl.pallas_call(
        _mid_kernel,
        grid=(N_BLKS,),
        out_shape=jax.ShapeDtypeStruct((N_NODES, HID), _f32),
        in_specs=[
            pl.BlockSpec((NC, ROW_BLK, HID), lambda i: (0, i, 0)),
            pl.BlockSpec((ROW_BLK, HID), lambda i: (i, 0)),
            pl.BlockSpec((ROW_BLK, NC), lambda i: (i, 0)),
            pl.BlockSpec((HID,), lambda i: (0,)),
            pl.BlockSpec((HID, HID), lambda i: (0, 0)),
        ],
        out_specs=pl.BlockSpec((ROW_BLK, HID), lambda i: (i, 0)),
    )(acc, hws, pdeg_t, b, w)


def _final_kernel(acc_ref, hws_ref, pdeg_ref, b_ref, wcls_ref, bcls_ref, bid_ref,
                  out_ref, sums_acc, cnts_acc):
    i = pl.program_id(0)

    @pl.when(i == 0)
    def _():
        sums_acc[...] = jnp.zeros_like(sums_acc)
        cnts_acc[...] = jnp.zeros_like(cnts_acc)

    t = acc_ref[0] + acc_ref[1] + hws_ref[...]
    h = _dinv_from_pdeg(pdeg_ref) * t + b_ref[...]
    y = jnp.dot(h, wcls_ref[...], preferred_element_type=_f32) + bcls_ref[...]

    bid = bid_ref[0]
    iota = lax.broadcasted_iota(_i32, (N_GROUPS, ROW_BLK), 0)
    mask = (bid == iota).astype(_f32)
    sums_acc[...] += jnp.dot(mask, y, preferred_element_type=_f32)
    cnts_acc[...] += jnp.broadcast_to(
        jnp.sum(mask, axis=1, keepdims=True), (N_GROUPS, D_OUT)
    )

    @pl.when(i == N_BLKS - 1)
    def _():
        out_ref[...] = sums_acc[...] / jnp.maximum(cnts_acc[...], 1.0)


def _tc_final(acc, hws, pdeg_t, b2, w_cls, b_cls, bid3):
    return pl.pallas_call(
        _final_kernel,
        grid=(N_BLKS,),
        out_shape=jax.ShapeDtypeStruct((N_GROUPS, D_OUT), _f32),
        in_specs=[
            pl.BlockSpec((NC, ROW_BLK, HID), lambda i: (0, i, 0)),
            pl.BlockSpec((ROW_BLK, HID), lambda i: (i, 0)),
            pl.BlockSpec((ROW_BLK, NC), lambda i: (i, 0)),
            pl.BlockSpec((HID,), lambda i: (0,)),
            pl.BlockSpec((HID, D_OUT), lambda i: (0, 0)),
            pl.BlockSpec((D_OUT,), lambda i: (0,)),
            pl.BlockSpec((1, 1, ROW_BLK), lambda i: (i, 0, 0)),
        ],
        out_specs=pl.BlockSpec((N_GROUPS, D_OUT), lambda i: (0, 0)),
        scratch_shapes=[
            pltpu.VMEM((N_GROUPS, D_OUT), _f32),
            pltpu.VMEM((N_GROUPS, D_OUT), _f32),
        ],
    )(acc, hws, pdeg_t, b2, w_cls, b_cls, bid3)


def kernel(x, edge_index, batch_ids, W_in, b_in, W0, b0, W1, b1, W2, b2,
           W_cls, b_cls):
    n_extra = E_PAD - N_EDGES
    pad_edges = jnp.stack([
        jnp.arange(n_extra, dtype=_i32) % N_NODES,
        N_NODES + (jnp.arange(n_extra, dtype=_i32) % N_DUMP),
    ])
    eidx = jnp.concatenate([edge_index, pad_edges], axis=1)

    bid3 = batch_ids.reshape(N_BLKS, 1, ROW_BLK)

    pdeg_t = _sc_degree(eidx).T

    hws0 = _tc_encode(x, W_in, b_in, W0, pdeg_t)
    acc0 = _sc_gather_scatter(hws0, eidx)
    hws1 = _tc_mid(acc0, hws0, pdeg_t, b0, W1)
    acc1 = _sc_gather_scatter(hws1, eidx)
    hws2 = _tc_mid(acc1, hws1, pdeg_t, b1, W2)
    acc2 = _sc_gather_scatter(hws2, eidx)
    return _tc_final(acc2, hws2, pdeg_t, b2, W_cls, b_cls, bid3)


# speedup vs baseline: 1.1761x; 1.0118x over previous
import functools

import jax
import jax.numpy as jnp
from jax import lax
from jax.experimental import pallas as pl
from jax.experimental.pallas import tpu as pltpu
from jax.experimental.pallas import tpu_sc as plsc

N_NODES = 10000
N_EDGES = 320000
D_IN = 128
HID = 128
D_OUT = 64
N_GROUPS = 16

NC = 2
NS = 16
NW = NC * NS
LANES = 16

N_PAD = 10240
ROWS_PER_TILE = N_PAD // NS
E_PAD = 327680
EPW = E_PAD // NW
KW = 64
NWIN = EPW // KW
CH = 40
NCH = NWIN // CH
SLOTS = 5
IW = 128
NROW = EPW // IW
N_DUMP = N_PAD - N_NODES

ROW_BLK = 5000
N_BLKS = N_NODES // ROW_BLK

_f32 = jnp.float32
_i32 = jnp.int32


def _sc_mesh():
    return plsc.VectorSubcoreMesh(
        core_axis_name="c", subcore_axis_name="s", num_cores=NC, num_subcores=NS
    )


def _zero_vec_ref(ref, nvecs):
    zeros16 = jnp.zeros((LANES,), _f32)

    def body(i, _):
        ref[pl.ds(i * LANES, LANES)] = zeros16
        return 0

    lax.fori_loop(0, nvecs, body, 0)


def _deg_body(eidx_hbm, out_hbm, didx_v, dsm_v, ones_v, zbuf_v, accd_sh, sem):
    del sem
    c = lax.axis_index("c")
    s = lax.axis_index("s")
    wid = c * NS + s

    pltpu.sync_copy(eidx_hbm.at[1].at[pl.ds(wid * EPW, EPW)], didx_v)

    ones16 = jnp.ones((LANES,), _f32)

    def fill_ones(i, _):
        ones_v[pl.ds(i * LANES, LANES)] = ones16
        return 0

    lax.fori_loop(0, IW // LANES, fill_ones, 0)
    _zero_vec_ref(zbuf_v, ROWS_PER_TILE // LANES)
    pltpu.sync_copy(zbuf_v, accd_sh.at[pl.ds(s * ROWS_PER_TILE, ROWS_PER_TILE)])
    plsc.subcore_barrier()

    def win(j, _):
        for k in range(IW // LANES):
            dsm_v[0, pl.ds(k * LANES, LANES)] = (
                didx_v[pl.ds(j * IW + k * LANES, LANES)])
        pltpu.sync_copy(ones_v, accd_sh.at[dsm_v.at[0]], add=True)
        return 0

    lax.fori_loop(0, NROW, win, 0)
    plsc.subcore_barrier()

    pltpu.sync_copy(
        accd_sh.at[pl.ds(s * ROWS_PER_TILE, ROWS_PER_TILE)],
        out_hbm.at[c].at[pl.ds(s * ROWS_PER_TILE, ROWS_PER_TILE)],
    )


def _sc_degree(eidx):
    k = pl.kernel(
        _deg_body,
        out_type=jax.ShapeDtypeStruct((NC, N_PAD), _f32),
        mesh=_sc_mesh(),
        scratch_types=[
            pltpu.VMEM((EPW,), _i32),
            pltpu.VMEM((1, IW), _i32),
            pltpu.VMEM((IW,), _f32),
            pltpu.VMEM((ROWS_PER_TILE,), _f32),
            pltpu.VMEM_SHARED((N_PAD,), _f32),
            pltpu.SemaphoreType.DMA,
        ],
        name="gcn_degree_sc",
    )
    return k(eidx)


def _scat_body(hws_hbm, eidx_hbm, out_hbm, sidx_v, didx_v, dsm_v, rows_v,
               acc_sh, sem0, sem1, sem2, sem3, sem4):
    c = lax.axis_index("c")
    s = lax.axis_index("s")
    wid = c * NS + s

    zrow = rows_v.at[0]

    def zrow_fill(i, _):
        zrow[i // (HID // LANES),
             pl.ds((i % (HID // LANES)) * LANES, LANES)] = jnp.zeros((LANES,), _f32)
        return 0

    lax.fori_loop(0, KW * HID // LANES, zrow_fill, 0)

    def zcopy(i, _):
        pltpu.sync_copy(zrow, acc_sh.at[pl.ds(s * ROWS_PER_TILE + i * KW, KW)])
        return 0

    lax.fori_loop(0, ROWS_PER_TILE // KW, zcopy, 0)
    plsc.subcore_barrier()

    sems = (sem0, sem1, sem2, sem3, sem4)
    cw = CH * KW

    def _sidx_win(j):
        return sidx_v.at[pl.ds(j * KW, KW)]

    def chunk(ch, _):
        base = wid * EPW + ch * cw
        pltpu.sync_copy(eidx_hbm.at[0].at[pl.ds(base, cw)], sidx_v)
        pltpu.sync_copy(eidx_hbm.at[1].at[pl.ds(base, cw)], didx_v)

        for b in range(SLOTS):
            pltpu.async_copy(hws_hbm.at[_sidx_win(b)], rows_v.at[b], sems[b])

        def win(w, _):
            for b in range(SLOTS):
                j = w * SLOTS + b
                buf = rows_v.at[b]
                pltpu.make_async_copy(hws_hbm.at[_sidx_win(j)], buf, sems[b]).wait()
                for k in range(KW // LANES):
                    dsm_v[b, pl.ds(k * LANES, LANES)] = (
                        didx_v[pl.ds(j * KW + k * LANES, LANES)])
                pltpu.sync_copy(buf, acc_sh.at[dsm_v.at[b]], add=True)

                @pl.when(j + SLOTS < CH)
                def _():
                    pltpu.async_copy(hws_hbm.at[_sidx_win(j + SLOTS)],
                                     buf, sems[b])
            return 0

        lax.fori_loop(0, CH // SLOTS, win, 0)
        return 0

    lax.fori_loop(0, NCH, chunk, 0)
    plsc.subcore_barrier()

    pltpu.sync_copy(
        acc_sh.at[pl.ds(s * ROWS_PER_TILE, ROWS_PER_TILE)],
        out_hbm.at[c].at[pl.ds(s * ROWS_PER_TILE, ROWS_PER_TILE)],
    )


def _sc_gather_scatter(hws, eidx):
    k = pl.kernel(
        _scat_body,
        out_type=jax.ShapeDtypeStruct((NC, N_PAD, HID), _f32),
        mesh=_sc_mesh(),
        scratch_types=[
            pltpu.VMEM((CH * KW,), _i32),
            pltpu.VMEM((CH * KW,), _i32),
            pltpu.VMEM((SLOTS, KW), _i32),
            pltpu.VMEM((SLOTS, KW, HID), _f32),
            pltpu.VMEM_SHARED((N_PAD, HID), _f32),
            pltpu.SemaphoreType.DMA,
            pltpu.SemaphoreType.DMA,
            pltpu.SemaphoreType.DMA,
            pltpu.SemaphoreType.DMA,
            pltpu.SemaphoreType.DMA,
        ],
        name="gcn_gather_scatter_sc",
    )
    return k(hws, eidx)


def _dinv_from_pdeg(pdeg_ref):
    deg = pdeg_ref[:, 0:1] + pdeg_ref[:, 1:2] + 1.0
    return 1.0 / jnp.sqrt(deg)


def _encode_kernel(x_ref, win_ref, bin_ref, w0_ref, pdeg_ref, out_ref):
    h0 = jnp.dot(x_ref[...], win_ref[...], preferred_element_type=_f32) + bin_ref[...]
    hw0 = jnp.dot(h0, w0_ref[...], preferred_element_type=_f32)
    out_ref[...] = _dinv_from_pdeg(pdeg_ref) * hw0


def _tc_encode(x, w_in, b_in, w0, pdeg_t):
    return pl.pallas_call(
        _encode_kernel,
        grid=(N_BLKS,),
        out_shape=jax.ShapeDtypeStruct((N_NODES, HID), _f32),
        in_specs=[
            pl.BlockSpec((ROW_BLK, D_IN), lambda i: (i, 0)),
            pl.BlockSpec((D_IN, HID), lambda i: (0, 0)),
            pl.BlockSpec((HID,), lambda i: (0,)),
            pl.BlockSpec((HID, HID), lambda i: (0, 0)),
            pl.BlockSpec((ROW_BLK, NC), lambda i: (i, 0)),
        ],
        out_specs=pl.BlockSpec((ROW_BLK, HID), lambda i: (i, 0)),
    )(x, w_in, b_in, w0, pdeg_t)


def _mid_kernel(acc_ref, hws_ref, pdeg_ref, b_ref, w_ref, out_ref):
    dinv = _dinv_from_pdeg(pdeg_ref)
    t = acc_ref[0] + acc_ref[1] + hws_ref[...]
    h = jnp.maximum(dinv * t + b_ref[...], 0.0)
    out_ref[...] = dinv * jnp.dot(h, w_ref[...], preferred_element_type=_f32)


def _tc_mid(acc, hws, pdeg_t, b, w):
    return pl.pallas_call(
        _mid_kernel,
        grid=(N_BLKS,),
        out_shape=jax.ShapeDtypeStruct((N_NODES, HID), _f32),
        in_specs=[
            pl.BlockSpec((NC, ROW_BLK, HID), lambda i: (0, i, 0)),
            pl.BlockSpec((ROW_BLK, HID), lambda i: (i, 0)),
            pl.BlockSpec((ROW_BLK, NC), lambda i: (i, 0)),
            pl.BlockSpec((HID,), lambda i: (0,)),
            pl.BlockSpec((HID, HID), lambda i: (0, 0)),
        ],
        out_specs=pl.BlockSpec((ROW_BLK, HID), lambda i: (i, 0)),
    )(acc, hws, pdeg_t, b, w)


def _final_kernel(acc_ref, hws_ref, pdeg_ref, b_ref, wcls_ref, bcls_ref, bid_ref,
                  out_ref, sums_acc, cnts_acc):
    i = pl.program_id(0)

    @pl.when(i == 0)
    def _():
        sums_acc[...] = jnp.zeros_like(sums_acc)
        cnts_acc[...] = jnp.zeros_like(cnts_acc)

    t = acc_ref[0] + acc_ref[1] + hws_ref[...]
    h = _dinv_from_pdeg(pdeg_ref) * t + b_ref[...]
    y = jnp.dot(h, wcls_ref[...], preferred_element_type=_f32) + bcls_ref[...]

    bid = bid_ref[0]
    iota = lax.broadcasted_iota(_i32, (N_GROUPS, ROW_BLK), 0)
    mask = (bid == iota).astype(_f32)
    sums_acc[...] += jnp.dot(mask, y, preferred_element_type=_f32)
    cnts_acc[...] += jnp.broadcast_to(
        jnp.sum(mask, axis=1, keepdims=True), (N_GROUPS, D_OUT)
    )

    @pl.when(i == N_BLKS - 1)
    def _():
        out_ref[...] = sums_acc[...] / jnp.maximum(cnts_acc[...], 1.0)


def _tc_final(acc, hws, pdeg_t, b2, w_cls, b_cls, bid3):
    return pl.pallas_call(
        _final_kernel,
        grid=(N_BLKS,),
        out_shape=jax.ShapeDtypeStruct((N_GROUPS, D_OUT), _f32),
        in_specs=[
            pl.BlockSpec((NC, ROW_BLK, HID), lambda i: (0, i, 0)),
            pl.BlockSpec((ROW_BLK, HID), lambda i: (i, 0)),
            pl.BlockSpec((ROW_BLK, NC), lambda i: (i, 0)),
            pl.BlockSpec((HID,), lambda i: (0,)),
            pl.BlockSpec((HID, D_OUT), lambda i: (0, 0)),
            pl.BlockSpec((D_OUT,), lambda i: (0,)),
            pl.BlockSpec((1, 1, ROW_BLK), lambda i: (i, 0, 0)),
        ],
        out_specs=pl.BlockSpec((N_GROUPS, D_OUT), lambda i: (0, 0)),
        scratch_shapes=[
            pltpu.VMEM((N_GROUPS, D_OUT), _f32),
            pltpu.VMEM((N_GROUPS, D_OUT), _f32),
        ],
    )(acc, hws, pdeg_t, b2, w_cls, b_cls, bid3)


def kernel(x, edge_index, batch_ids, W_in, b_in, W0, b0, W1, b1, W2, b2,
           W_cls, b_cls):
    n_extra = E_PAD - N_EDGES
    pad_edges = jnp.stack([
        jnp.arange(n_extra, dtype=_i32) % N_NODES,
        N_NODES + (jnp.arange(n_extra, dtype=_i32) % N_DUMP),
    ])
    eidx = jnp.concatenate([edge_index, pad_edges], axis=1)

    bid3 = batch_ids.reshape(N_BLKS, 1, ROW_BLK)

    pdeg_t = _sc_degree(eidx).T

    hws0 = _tc_encode(x, W_in, b_in, W0, pdeg_t)
    acc0 = _sc_gather_scatter(hws0, eidx)
    hws1 = _tc_mid(acc0, hws0, pdeg_t, b0, W1)
    acc1 = _sc_gather_scatter(hws1, eidx)
    hws2 = _tc_mid(acc1, hws1, pdeg_t, b1, W2)
    acc2 = _sc_gather_scatter(hws2, eidx)
    return _tc_final(acc2, hws2, pdeg_t, b2, W_cls, b_cls, bid3)
